# baseline (speedup 1.0000x reference)
import functools

import jax
import jax.numpy as jnp
from jax import lax
from jax.experimental import pallas as pl
from jax.experimental.pallas import tpu as pltpu
from jax.experimental.pallas import tpu_sc as plsc

N = 10000
E = 320000
D = 128
N_PAD = 10240
ROWS_PER_SUB = 640
CHUNK = 80
NEG_SLOPE = 0.01
DEG_W = 16
NB = 1000
GRID_N = N // NB


def _sc_mesh():
    return plsc.VectorSubcoreMesh(core_axis_name="c", subcore_axis_name="s")


def _deg_body(e0s, e0d, e1s, e1d, e2s, e2d, e3s, e3d, out,
              cnt0, cnt1, cnt2, cnt3, cnt4, cnt5, cnt6, cnt7, idx_v, sem):
    del sem
    c = lax.axis_index("c")
    s = lax.axis_index("s")
    wid = c * 16 + s
    cnts = (cnt0, cnt1, cnt2, cnt3, cnt4, cnt5, cnt6, cnt7)
    zeros16 = jnp.zeros((16,), jnp.float32)

    def zbody(i, carry):
        for cnt in cnts:
            cnt[pl.ds(i * 16, 16)] = zeros16
        return carry
    lax.fori_loop(0, N_PAD // 16, zbody, 0)

    ones16 = jnp.full((16,), 1.0, jnp.float32)
    streams = (e0s, e0d, e1s, e1d, e2s, e2d, e3s, e3d)
    per_sub = E // 32

    for k in range(8):
        pltpu.sync_copy(streams[k].at[pl.ds(wid * per_sub, per_sub)], idx_v)

        def body(i, carry, k=k):
            idx16 = idx_v[pl.ds(i * 16, 16)]
            plsc.addupdate_scatter(cnts[k], [idx16], ones16)
            return carry
        lax.fori_loop(0, per_sub // 16, body, 0)

    for k in range(8):
        pltpu.sync_copy(cnts[k], out.at[pl.ds((wid * 8 + k) * N_PAD, N_PAD)])


def _degrees(srcs, dsts):
    fn = pl.kernel(
        _deg_body,
        out_type=jax.ShapeDtypeStruct((32 * 8 * N_PAD,), jnp.float32),
        mesh=_sc_mesh(),
        compiler_params=pltpu.CompilerParams(needs_layout_passes=False),
        scratch_types=[
            *([pltpu.VMEM((N_PAD,), jnp.float32)] * 8),
            pltpu.VMEM((E // 32,), jnp.int32),
            pltpu.SemaphoreType.DMA,
        ],
    )
    return fn(srcs[0], dsts[0], srcs[1], dsts[1], srcs[2], dsts[2],
              srcs[3], dsts[3])


def _degred_tc(x_ref, o_ref):
    acc = x_ref[0:8, :]
    for t in range(1, 32):
        acc = acc + x_ref[t * 8:(t + 1) * 8, :]
    o_ref[...] = acc


def _degred(deg_flat):
    x = deg_flat.reshape(32 * 8, N_PAD)
    return pl.pallas_call(
        _degred_tc,
        grid=(N_PAD // 128,),
        in_specs=[pl.BlockSpec((32 * 8, 128), lambda i: (0, i))],
        out_specs=pl.BlockSpec((8, 128), lambda i: (0, i)),
        out_shape=jax.ShapeDtypeStruct((8, N_PAD), jnp.float32),
    )(x)


AGG_CHUNK = 128
AGG_NCH = -(-(E // 32) // AGG_CHUNK)
AGG_PS = (AGG_NCH + 1) * AGG_CHUNK


def _agg_body(h, src, dst, zeros, out, acc,
              ix0, ix1, dx0, dx1, rows0, rows1, is0, is1, gs0, gs1):
    c = lax.axis_index("c")
    s = lax.axis_index("s")
    base = (c * 16 + s) * AGG_PS
    pltpu.sync_copy(zeros, acc.at[pl.ds(s * ROWS_PER_SUB, ROWS_PER_SUB)])
    plsc.subcore_barrier()

    ix = (ix0, ix1)
    dx = (dx0, dx1)
    rows = (rows0, rows1)
    isem = (is0, is1)
    gsem = (gs0, gs1)

    def islice(j):
        return pl.ds(base + j * AGG_CHUNK, AGG_CHUNK)

    def idx_start(slot, j):
        pltpu.async_copy(src.at[islice(j)], ix[slot], isem[slot])
        pltpu.async_copy(dst.at[islice(j)], dx[slot], isem[slot])

    def idx_wait(slot, j):
        pltpu.make_async_copy(src.at[islice(j)], ix[slot], isem[slot]).wait()
        pltpu.make_async_copy(dst.at[islice(j)], dx[slot], isem[slot]).wait()

    def g_start(slot):
        pltpu.async_copy(h.at[ix[slot]], rows[slot], gsem[slot])

    def fin(slot):
        pltpu.make_async_copy(h.at[ix[slot]], rows[slot], gsem[slot]).wait()
        pltpu.sync_copy(rows[slot], acc.at[dx[slot]], add=True)

    idx_start(0, 0)
    idx_wait(0, 0)
    g_start(0)
    idx_start(1, 1)

    def phase(j, slot):
        nxt = 1 - slot
        idx_wait(nxt, j + 1)
        g_start(nxt)
        fin(slot)
        idx_start(slot, j + 2)

    def body(i, carry):
        phase(2 * i, 0)
        phase(2 * i + 1, 1)
        return carry
    lax.fori_loop(0, (AGG_NCH - 1) // 2, body, 0)

    fin(0)
    idx_wait(1, AGG_NCH)

    plsc.subcore_barrier()
    pltpu.sync_copy(acc.at[pl.ds(s * ROWS_PER_SUB, ROWS_PER_SUB)],
                    out.at[c, pl.ds(s * ROWS_PER_SUB, ROWS_PER_SUB)])


def _aggregate(h, src_p, dst_p):
    zeros = jnp.zeros((ROWS_PER_SUB, D), jnp.float32)
    fn = pl.kernel(
        _agg_body,
        out_type=jax.ShapeDtypeStruct((2, N_PAD, D), jnp.float32),
        mesh=_sc_mesh(),
        scratch_types=[
            pltpu.VMEM_SHARED((N_PAD, D), jnp.float32),
            pltpu.VMEM((AGG_CHUNK,), jnp.int32),
            pltpu.VMEM((AGG_CHUNK,), jnp.int32),
            pltpu.VMEM((AGG_CHUNK,), jnp.int32),
            pltpu.VMEM((AGG_CHUNK,), jnp.int32),
            pltpu.VMEM((AGG_CHUNK, D), jnp.float32),
            pltpu.VMEM((AGG_CHUNK, D), jnp.float32),
            pltpu.SemaphoreType.DMA,
            pltpu.SemaphoreType.DMA,
            pltpu.SemaphoreType.DMA,
            pltpu.SemaphoreType.DMA,
        ],
    )
    return fn(h, src_p, dst_p, zeros)


def _prep_tc(deg_ref, f_ref, rs_ref, norm_ref, h0_0, h0_1, h0_2, h0_3):
    deg = deg_ref[...]
    norm = jnp.where(deg > 0.0, lax.rsqrt(jnp.maximum(deg, 1.0)), 0.0)
    norm_ref[...] = norm
    f = f_ref[...]
    for r, href in enumerate((h0_0, h0_1, h0_2, h0_3)):
        href[...] = f * rs_ref[r, :][None, :] * norm[:, 2 * r][:, None]


def _prep(degs, features, rs_pad):
    out_shape = (
        jax.ShapeDtypeStruct((N, 8), jnp.float32),
        *([jax.ShapeDtypeStruct((N, D), jnp.float32)] * 4),
    )
    return pl.pallas_call(
        _prep_tc,
        grid=(GRID_N,),
        in_specs=[
            pl.BlockSpec((NB, 8), lambda i: (i, 0)),
            pl.BlockSpec((NB, D), lambda i: (i, 0)),
            pl.BlockSpec((8, D), lambda i: (0, 0)),
        ],
        out_specs=(
            pl.BlockSpec((NB, 8), lambda i: (i, 0)),
            *([pl.BlockSpec((NB, D), lambda i: (i, 0))] * 4),
        ),
        out_shape=out_shape,
    )(degs, features, rs_pad)


def _dense_tc_stats(p_ref, nd_ref, w_ref, b_ref, y_ref, st_ref):
    agg = (p_ref[0] + p_ref[1]) * nd_ref[...]
    y = jnp.dot(agg, w_ref[...], preferred_element_type=jnp.float32) \
        + b_ref[0, :][None, :]
    y_ref[...] = y

    @pl.when(pl.program_id(0) == 0)
    def _():
        st_ref[...] = jnp.zeros_like(st_ref)
    st_ref[0:1, :] += jnp.sum(y, axis=0, keepdims=True)
    st_ref[1:2, :] += jnp.sum(y * y, axis=0, keepdims=True)


def _dense_tc(p_ref, nd_ref, w_ref, b_ref, y_ref):
    agg = (p_ref[0] + p_ref[1]) * nd_ref[...]
    y_ref[...] = jnp.dot(agg, w_ref[...], preferred_element_type=jnp.float32) \
        + b_ref[0, :][None, :]


def _dense(parts, norm_dst, w, b, with_stats):
    in_specs = [
        pl.BlockSpec((2, NB, D), lambda i: (0, i, 0)),
        pl.BlockSpec((NB, 1), lambda i: (i, 0)),
        pl.BlockSpec((D, D), lambda i: (0, 0)),
        pl.BlockSpec((1, D), lambda i: (0, 0)),
    ]
    if with_stats:
        return pl.pallas_call(
            _dense_tc_stats,
            grid=(GRID_N,),
            in_specs=in_specs,
            out_specs=(
                pl.BlockSpec((NB, D), lambda i: (i, 0)),
                pl.BlockSpec((8, D), lambda i: (0, 0)),
            ),
            out_shape=(
                jax.ShapeDtypeStruct((N, D), jnp.float32),
                jax.ShapeDtypeStruct((8, D), jnp.float32),
            ),
        )(parts, norm_dst, w, b[None, :])
    return pl.pallas_call(
        _dense_tc,
        grid=(GRID_N,),
        in_specs=in_specs,
        out_specs=pl.BlockSpec((NB, D), lambda i: (i, 0)),
        out_shape=jax.ShapeDtypeStruct((N, D), jnp.float32),
    )(parts, norm_dst, w, b[None, :])


def _post_tc(y_ref, st_ref, f_ref, r1_ref, ns_ref, g_ref, bt_ref, h1_ref):
    mean = st_ref[0:1, :] / N
    var = st_ref[1:2, :] / N - mean * mean
    z = (y_ref[...] - mean) * lax.rsqrt(var + 1e-5) * g_ref[0:1, :] \
        + bt_ref[0:1, :]
    z = jnp.where(z >= 0.0, z, NEG_SLOPE * z)
    emb = f_ref[...] + z
    h1_ref[...] = emb * r1_ref[0:1, :] * ns_ref[...]


def _post(y, stats, features, r1_row, norm_src, gamma, beta):
    return pl.pallas_call(
        _post_tc,
        grid=(GRID_N,),
        in_specs=[
            pl.BlockSpec((NB, D), lambda i: (i, 0)),
            pl.BlockSpec((8, D), lambda i: (0, 0)),
            pl.BlockSpec((NB, D), lambda i: (i, 0)),
            pl.BlockSpec((1, D), lambda i: (0, 0)),
            pl.BlockSpec((NB, 1), lambda i: (i, 0)),
            pl.BlockSpec((1, D), lambda i: (0, 0)),
            pl.BlockSpec((1, D), lambda i: (0, 0)),
        ],
        out_specs=pl.BlockSpec((NB, D), lambda i: (i, 0)),
        out_shape=jax.ShapeDtypeStruct((N, D), jnp.float32),
    )(y, stats, features, r1_row, norm_src, gamma[None, :], beta[None, :])


def _relvec_tc(rs_ref, w0_ref, b0_ref, w1_ref, b1_ref, r1_ref, r2_ref):
    r1 = jnp.dot(rs_ref[...], w0_ref[...], preferred_element_type=jnp.float32) \
        + b0_ref[0:1, :]
    r2 = jnp.dot(r1, w1_ref[...], preferred_element_type=jnp.float32) \
        + b1_ref[0:1, :]
    r1_ref[...] = r1
    r2_ref[...] = r2


def _relvec(rs_pad, w0, b0, w1, b1):
    return pl.pallas_call(
        _relvec_tc,
        out_shape=(
            jax.ShapeDtypeStruct((8, D), jnp.float32),
            jax.ShapeDtypeStruct((8, D), jnp.float32),
        ),
    )(rs_pad, w0, b0[None, :], w1, b1[None, :])


def kernel(features, poi_r, s_r, d_r, n_r,
           poi_edge_index, s_edge_index, d_edge_index, n_edge_index,
           W_gcn0, b_gcn0, W_gcn1, b_gcn1,
           bn_gamma0, bn_beta0,
           W_rel0, b_rel0, W_rel1, b_rel1):
    edges = [n_edge_index, poi_edge_index, s_edge_index, d_edge_index]
    srcs = [e[0] for e in edges]
    dsts = [e[1] for e in edges]
    def _pad_idx(x, fill):
        return jnp.pad(x.reshape(32, E // 32),
                       ((0, 0), (0, AGG_PS - E // 32)),
                       constant_values=fill).reshape(-1)
    srcs_p = [_pad_idx(x, 0) for x in srcs]
    dsts_p = [_pad_idx(x, N_PAD - 1) for x in dsts]
    rs_pad = jnp.concatenate(
        [jnp.stack([n_r, poi_r, s_r, d_r]), jnp.zeros((4, D), jnp.float32)], axis=0)

    deg_flat = _degrees(srcs, dsts)
    deg8 = _degred(deg_flat)
    degs = deg8[:, :N].transpose(1, 0)

    r1_pad, r2_pad = _relvec(rs_pad, W_rel0, b_rel0, W_rel1, b_rel1)
    norms, h0_0, h0_1, h0_2, h0_3 = _prep(degs, features, rs_pad)
    h0s = (h0_0, h0_1, h0_2, h0_3)

    embs = []
    for r in range(4):
        nd = norms[:, 2 * r + 1:2 * r + 2]
        ns = norms[:, 2 * r:2 * r + 1]
        parts0 = _aggregate(h0s[r], srcs_p[r], dsts_p[r])
        y, stats = _dense(parts0, nd, W_gcn0, b_gcn0, True)
        h1 = _post(y, stats, features, r1_pad[r:r + 1, :], ns,
                   bn_gamma0, bn_beta0)
        parts1 = _aggregate(h1, srcs_p[r], dsts_p[r])
        embs.append(_dense(parts1, nd, W_gcn1, b_gcn1, False))

    return (embs[0], embs[1], embs[2], embs[3],
            r2_pad[0], r2_pad[1], r2_pad[2], r2_pad[3])


# speedup vs baseline: 1.0297x; 1.0297x over previous
import functools

import jax
import jax.numpy as jnp
from jax import lax
from jax.experimental import pallas as pl
from jax.experimental.pallas import tpu as pltpu
from jax.experimental.pallas import tpu_sc as plsc

N = 10000
E = 320000
D = 128
N_PAD = 10240
ROWS_PER_SUB = 640
CHUNK = 80
NEG_SLOPE = 0.01
DEG_W = 16
NB = 1000
GRID_N = N // NB


def _sc_mesh():
    return plsc.VectorSubcoreMesh(core_axis_name="c", subcore_axis_name="s")


def _deg_body(e0s, e0d, e1s, e1d, e2s, e2d, e3s, e3d, out,
              cnt0, cnt1, cnt2, cnt3, cnt4, cnt5, cnt6, cnt7, idx_v, sem):
    del sem
    c = lax.axis_index("c")
    s = lax.axis_index("s")
    wid = c * 16 + s
    cnts = (cnt0, cnt1, cnt2, cnt3, cnt4, cnt5, cnt6, cnt7)
    zeros16 = jnp.zeros((16,), jnp.float32)

    def zbody(i, carry):
        for cnt in cnts:
            cnt[pl.ds(i * 16, 16)] = zeros16
        return carry
    lax.fori_loop(0, N_PAD // 16, zbody, 0)

    ones16 = jnp.full((16,), 1.0, jnp.float32)
    streams = (e0s, e0d, e1s, e1d, e2s, e2d, e3s, e3d)
    per_sub = E // 32

    for k in range(8):
        pltpu.sync_copy(streams[k].at[pl.ds(wid * per_sub, per_sub)], idx_v)

        def body(i, carry, k=k):
            idx16 = idx_v[pl.ds(i * 16, 16)]
            plsc.addupdate_scatter(cnts[k], [idx16], ones16)
            return carry
        lax.fori_loop(0, per_sub // 16, body, 0)

    for k in range(8):
        pltpu.sync_copy(cnts[k], out.at[pl.ds((wid * 8 + k) * N_PAD, N_PAD)])


def _degrees(srcs, dsts):
    fn = pl.kernel(
        _deg_body,
        out_type=jax.ShapeDtypeStruct((32 * 8 * N_PAD,), jnp.float32),
        mesh=_sc_mesh(),
        compiler_params=pltpu.CompilerParams(needs_layout_passes=False),
        scratch_types=[
            *([pltpu.VMEM((N_PAD,), jnp.float32)] * 8),
            pltpu.VMEM((E // 32,), jnp.int32),
            pltpu.SemaphoreType.DMA,
        ],
    )
    return fn(srcs[0], dsts[0], srcs[1], dsts[1], srcs[2], dsts[2],
              srcs[3], dsts[3])


def _degred_tc(x_ref, o_ref):
    acc = x_ref[0:8, :]
    for t in range(1, 32):
        acc = acc + x_ref[t * 8:(t + 1) * 8, :]
    o_ref[...] = acc


def _degred(deg_flat):
    x = deg_flat.reshape(32 * 8, N_PAD)
    return pl.pallas_call(
        _degred_tc,
        grid=(N_PAD // 128,),
        in_specs=[pl.BlockSpec((32 * 8, 128), lambda i: (0, i))],
        out_specs=pl.BlockSpec((8, 128), lambda i: (0, i)),
        out_shape=jax.ShapeDtypeStruct((8, N_PAD), jnp.float32),
    )(x)


AGG_CHUNK = 80
AGG_NCH = 126
AGG_PS = AGG_NCH * AGG_CHUNK


def _agg_body(h, src, dst, zeros, out, acc,
              ix0, ix1, ix2, dx0, dx1, dx2, rows0, rows1,
              is0, is1, is2, gs0, gs1, ss0, ss1):
    c = lax.axis_index("c")
    s = lax.axis_index("s")
    base = (c * 16 + s) * AGG_PS
    pltpu.sync_copy(zeros, acc.at[pl.ds(s * ROWS_PER_SUB, ROWS_PER_SUB)])
    plsc.subcore_barrier()

    ix = (ix0, ix1, ix2)
    dx = (dx0, dx1, dx2)
    rows = (rows0, rows1)
    isem = (is0, is1, is2)
    gsem = (gs0, gs1)
    ssem = (ss0, ss1)

    def islice(j):
        return pl.ds(base + j * AGG_CHUNK, AGG_CHUNK)

    def idx_start(k, j):
        pltpu.async_copy(src.at[islice(j)], ix[k], isem[k])
        pltpu.async_copy(dst.at[islice(j)], dx[k], isem[k])

    def idx_wait(k, j):
        pltpu.make_async_copy(src.at[islice(j)], ix[k], isem[k]).wait()
        pltpu.make_async_copy(dst.at[islice(j)], dx[k], isem[k]).wait()

    def g_start(r, k):
        pltpu.async_copy(h.at[ix[k]], rows[r], gsem[r])

    def g_wait(r, k):
        pltpu.make_async_copy(h.at[ix[k]], rows[r], gsem[r]).wait()

    def s_start(r, k):
        pltpu.async_copy(rows[r], acc.at[dx[k]], ssem[r], add=True)

    def s_wait(r, k):
        pltpu.make_async_copy(rows[r], acc.at[dx[k]], ssem[r]).wait()

    idx_start(0, 0)
    idx_wait(0, 0)
    g_start(0, 0)
    idx_start(1, 1)

    n_it = AGG_NCH // 6

    def body(i, carry):
        for b in range(6):
            j = 6 * i + b
            rs, rn = b % 2, 1 - b % 2
            kc = b % 3
            kn = (b + 1) % 3
            kp = (b + 2) % 3
            g_wait(rs, kc)
            s_start(rs, kc)
            if b == 0:
                @pl.when(i > 0)
                def _():
                    s_wait(rn, kp)
            else:
                s_wait(rn, kp)
            if b >= 4:
                @pl.when(i < n_it - 1)
                def _(j=j, kp=kp):
                    idx_start(kp, j + 2)
            else:
                idx_start(kp, j + 2)
            if b == 5:
                @pl.when(i < n_it - 1)
                def _(j=j, kn=kn, rn=rn):
                    idx_wait(kn, j + 1)
                    g_start(rn, kn)
            else:
                idx_wait(kn, j + 1)
                g_start(rn, kn)
        return carry
    lax.fori_loop(0, n_it, body, 0)
    s_wait(1, (AGG_NCH - 1) % 3)

    plsc.subcore_barrier()
    pltpu.sync_copy(acc.at[pl.ds(s * ROWS_PER_SUB, ROWS_PER_SUB)],
                    out.at[c, pl.ds(s * ROWS_PER_SUB, ROWS_PER_SUB)])


def _aggregate(h, src_p, dst_p):
    zeros = jnp.zeros((ROWS_PER_SUB, D), jnp.float32)
    fn = pl.kernel(
        _agg_body,
        out_type=jax.ShapeDtypeStruct((2, N_PAD, D), jnp.float32),
        mesh=_sc_mesh(),
        scratch_types=[
            pltpu.VMEM_SHARED((N_PAD, D), jnp.float32),
            pltpu.VMEM((AGG_CHUNK,), jnp.int32),
            pltpu.VMEM((AGG_CHUNK,), jnp.int32),
            pltpu.VMEM((AGG_CHUNK,), jnp.int32),
            pltpu.VMEM((AGG_CHUNK,), jnp.int32),
            pltpu.VMEM((AGG_CHUNK,), jnp.int32),
            pltpu.VMEM((AGG_CHUNK,), jnp.int32),
            pltpu.VMEM((AGG_CHUNK, D), jnp.float32),
            pltpu.VMEM((AGG_CHUNK, D), jnp.float32),
            pltpu.SemaphoreType.DMA,
            pltpu.SemaphoreType.DMA,
            pltpu.SemaphoreType.DMA,
            pltpu.SemaphoreType.DMA,
            pltpu.SemaphoreType.DMA,
            pltpu.SemaphoreType.DMA,
            pltpu.SemaphoreType.DMA,
        ],
    )
    return fn(h, src_p, dst_p, zeros)


def _prep_tc(deg_ref, f_ref, rs_ref, norm_ref, h0_0, h0_1, h0_2, h0_3):
    deg = deg_ref[...]
    norm = jnp.where(deg > 0.0, lax.rsqrt(jnp.maximum(deg, 1.0)), 0.0)
    norm_ref[...] = norm
    f = f_ref[...]
    for r, href in enumerate((h0_0, h0_1, h0_2, h0_3)):
        href[...] = f * rs_ref[r, :][None, :] * norm[:, 2 * r][:, None]


def _prep(degs, features, rs_pad):
    out_shape = (
        jax.ShapeDtypeStruct((N, 8), jnp.float32),
        *([jax.ShapeDtypeStruct((N, D), jnp.float32)] * 4),
    )
    return pl.pallas_call(
        _prep_tc,
        grid=(GRID_N,),
        in_specs=[
            pl.BlockSpec((NB, 8), lambda i: (i, 0)),
            pl.BlockSpec((NB, D), lambda i: (i, 0)),
            pl.BlockSpec((8, D), lambda i: (0, 0)),
        ],
        out_specs=(
            pl.BlockSpec((NB, 8), lambda i: (i, 0)),
            *([pl.BlockSpec((NB, D), lambda i: (i, 0))] * 4),
        ),
        out_shape=out_shape,
    )(degs, features, rs_pad)


def _dense_tc_stats(p_ref, nd_ref, w_ref, b_ref, y_ref, st_ref):
    agg = (p_ref[0] + p_ref[1]) * nd_ref[...]
    y = jnp.dot(agg, w_ref[...], preferred_element_type=jnp.float32) \
        + b_ref[0, :][None, :]
    y_ref[...] = y

    @pl.when(pl.program_id(0) == 0)
    def _():
        st_ref[...] = jnp.zeros_like(st_ref)
    st_ref[0:1, :] += jnp.sum(y, axis=0, keepdims=True)
    st_ref[1:2, :] += jnp.sum(y * y, axis=0, keepdims=True)


def _dense_tc(p_ref, nd_ref, w_ref, b_ref, y_ref):
    agg = (p_ref[0] + p_ref[1]) * nd_ref[...]
    y_ref[...] = jnp.dot(agg, w_ref[...], preferred_element_type=jnp.float32) \
        + b_ref[0, :][None, :]


def _dense(parts, norm_dst, w, b, with_stats):
    in_specs = [
        pl.BlockSpec((2, NB, D), lambda i: (0, i, 0)),
        pl.BlockSpec((NB, 1), lambda i: (i, 0)),
        pl.BlockSpec((D, D), lambda i: (0, 0)),
        pl.BlockSpec((1, D), lambda i: (0, 0)),
    ]
    if with_stats:
        return pl.pallas_call(
            _dense_tc_stats,
            grid=(GRID_N,),
            in_specs=in_specs,
            out_specs=(
                pl.BlockSpec((NB, D), lambda i: (i, 0)),
                pl.BlockSpec((8, D), lambda i: (0, 0)),
            ),
            out_shape=(
                jax.ShapeDtypeStruct((N, D), jnp.float32),
                jax.ShapeDtypeStruct((8, D), jnp.float32),
            ),
        )(parts, norm_dst, w, b[None, :])
    return pl.pallas_call(
        _dense_tc,
        grid=(GRID_N,),
        in_specs=in_specs,
        out_specs=pl.BlockSpec((NB, D), lambda i: (i, 0)),
        out_shape=jax.ShapeDtypeStruct((N, D), jnp.float32),
    )(parts, norm_dst, w, b[None, :])


def _post_tc(y_ref, st_ref, f_ref, r1_ref, ns_ref, g_ref, bt_ref, h1_ref):
    mean = st_ref[0:1, :] / N
    var = st_ref[1:2, :] / N - mean * mean
    z = (y_ref[...] - mean) * lax.rsqrt(var + 1e-5) * g_ref[0:1, :] \
        + bt_ref[0:1, :]
    z = jnp.where(z >= 0.0, z, NEG_SLOPE * z)
    emb = f_ref[...] + z
    h1_ref[...] = emb * r1_ref[0:1, :] * ns_ref[...]


def _post(y, stats, features, r1_row, norm_src, gamma, beta):
    return pl.pallas_call(
        _post_tc,
        grid=(GRID_N,),
        in_specs=[
            pl.BlockSpec((NB, D), lambda i: (i, 0)),
            pl.BlockSpec((8, D), lambda i: (0, 0)),
            pl.BlockSpec((NB, D), lambda i: (i, 0)),
            pl.BlockSpec((1, D), lambda i: (0, 0)),
            pl.BlockSpec((NB, 1), lambda i: (i, 0)),
            pl.BlockSpec((1, D), lambda i: (0, 0)),
            pl.BlockSpec((1, D), lambda i: (0, 0)),
        ],
        out_specs=pl.BlockSpec((NB, D), lambda i: (i, 0)),
        out_shape=jax.ShapeDtypeStruct((N, D), jnp.float32),
    )(y, stats, features, r1_row, norm_src, gamma[None, :], beta[None, :])


def _relvec_tc(rs_ref, w0_ref, b0_ref, w1_ref, b1_ref, r1_ref, r2_ref):
    r1 = jnp.dot(rs_ref[...], w0_ref[...], preferred_element_type=jnp.float32) \
        + b0_ref[0:1, :]
    r2 = jnp.dot(r1, w1_ref[...], preferred_element_type=jnp.float32) \
        + b1_ref[0:1, :]
    r1_ref[...] = r1
    r2_ref[...] = r2


def _relvec(rs_pad, w0, b0, w1, b1):
    return pl.pallas_call(
        _relvec_tc,
        out_shape=(
            jax.ShapeDtypeStruct((8, D), jnp.float32),
            jax.ShapeDtypeStruct((8, D), jnp.float32),
        ),
    )(rs_pad, w0, b0[None, :], w1, b1[None, :])


def kernel(features, poi_r, s_r, d_r, n_r,
           poi_edge_index, s_edge_index, d_edge_index, n_edge_index,
           W_gcn0, b_gcn0, W_gcn1, b_gcn1,
           bn_gamma0, bn_beta0,
           W_rel0, b_rel0, W_rel1, b_rel1):
    edges = [n_edge_index, poi_edge_index, s_edge_index, d_edge_index]
    srcs = [e[0] for e in edges]
    dsts = [e[1] for e in edges]
    def _pad_idx(x, fill):
        return jnp.pad(x.reshape(32, E // 32),
                       ((0, 0), (0, AGG_PS - E // 32)),
                       constant_values=fill).reshape(-1)
    srcs_p = [_pad_idx(x, 0) for x in srcs]
    dsts_p = [_pad_idx(x, N_PAD - 1) for x in dsts]
    rs_pad = jnp.concatenate(
        [jnp.stack([n_r, poi_r, s_r, d_r]), jnp.zeros((4, D), jnp.float32)], axis=0)

    deg_flat = _degrees(srcs, dsts)
    deg8 = _degred(deg_flat)
    degs = deg8[:, :N].transpose(1, 0)

    r1_pad, r2_pad = _relvec(rs_pad, W_rel0, b_rel0, W_rel1, b_rel1)
    norms, h0_0, h0_1, h0_2, h0_3 = _prep(degs, features, rs_pad)
    h0s = (h0_0, h0_1, h0_2, h0_3)

    embs = []
    for r in range(4):
        nd = norms[:, 2 * r + 1:2 * r + 2]
        ns = norms[:, 2 * r:2 * r + 1]
        parts0 = _aggregate(h0s[r], srcs_p[r], dsts_p[r])
        y, stats = _dense(parts0, nd, W_gcn0, b_gcn0, True)
        h1 = _post(y, stats, features, r1_pad[r:r + 1, :], ns,
                   bn_gamma0, bn_beta0)
        parts1 = _aggregate(h1, srcs_p[r], dsts_p[r])
        embs.append(_dense(parts1, nd, W_gcn1, b_gcn1, False))

    return (embs[0], embs[1], embs[2], embs[3],
            r2_pad[0], r2_pad[1], r2_pad[2], r2_pad[3])


# speedup vs baseline: 1.5038x; 1.4604x over previous
import functools

import jax
import jax.numpy as jnp
from jax import lax
from jax.experimental import pallas as pl
from jax.experimental.pallas import tpu as pltpu
from jax.experimental.pallas import tpu_sc as plsc

N = 10000
E = 320000
D = 128
N_PAD = 10240
ROWS_PER_SUB = 640
CHUNK = 80
NEG_SLOPE = 0.01
DEG_W = 16
NB = 1000
GRID_N = N // NB


def _sc_mesh():
    return plsc.VectorSubcoreMesh(core_axis_name="c", subcore_axis_name="s")


def _deg_body(e0s, e0d, e1s, e1d, e2s, e2d, e3s, e3d, out,
              cnt0, cnt1, cnt2, cnt3, cnt4, cnt5, cnt6, cnt7, idx_v, sem):
    del sem
    c = lax.axis_index("c")
    s = lax.axis_index("s")
    wid = c * 16 + s
    cnts = (cnt0, cnt1, cnt2, cnt3, cnt4, cnt5, cnt6, cnt7)
    zeros16 = jnp.zeros((16,), jnp.float32)

    def zbody(i, carry):
        for cnt in cnts:
            cnt[pl.ds(i * 16, 16)] = zeros16
        return carry
    lax.fori_loop(0, N_PAD // 16, zbody, 0)

    ones16 = jnp.full((16,), 1.0, jnp.float32)
    streams = (e0s, e0d, e1s, e1d, e2s, e2d, e3s, e3d)
    per_sub = E // 32

    for k in range(8):
        pltpu.sync_copy(streams[k].at[pl.ds(wid * per_sub, per_sub)], idx_v)

        def body(i, carry, k=k):
            idx16 = idx_v[pl.ds(i * 16, 16)]
            plsc.addupdate_scatter(cnts[k], [idx16], ones16)
            return carry
        lax.fori_loop(0, per_sub // 16, body, 0)

    for k in range(8):
        pltpu.sync_copy(cnts[k], out.at[pl.ds((wid * 8 + k) * N_PAD, N_PAD)])


def _degrees(srcs, dsts):
    fn = pl.kernel(
        _deg_body,
        out_type=jax.ShapeDtypeStruct((32 * 8 * N_PAD,), jnp.float32),
        mesh=_sc_mesh(),
        compiler_params=pltpu.CompilerParams(needs_layout_passes=False),
        scratch_types=[
            *([pltpu.VMEM((N_PAD,), jnp.float32)] * 8),
            pltpu.VMEM((E // 32,), jnp.int32),
            pltpu.SemaphoreType.DMA,
        ],
    )
    return fn(srcs[0], dsts[0], srcs[1], dsts[1], srcs[2], dsts[2],
              srcs[3], dsts[3])


def _degred_tc(x_ref, o_ref):
    acc = x_ref[0:8, :]
    for t in range(1, 32):
        acc = acc + x_ref[t * 8:(t + 1) * 8, :]
    o_ref[...] = acc


def _degred(deg_flat):
    x = deg_flat.reshape(32 * 8, N_PAD)
    return pl.pallas_call(
        _degred_tc,
        grid=(N_PAD // 128,),
        in_specs=[pl.BlockSpec((32 * 8, 128), lambda i: (0, i))],
        out_specs=pl.BlockSpec((8, 128), lambda i: (0, i)),
        out_shape=jax.ShapeDtypeStruct((8, N_PAD), jnp.float32),
    )(x)


AGG_CHUNK = 80
AGG_NCH = 126
AGG_PS = AGG_NCH * AGG_CHUNK


def _agg_body(h, src, dst, zeros, out, acc,
              ix0, ix1, ix2, dx0, dx1, dx2, rows0, rows1,
              is0, is1, is2, gs0, gs1, ss0, ss1):
    c = lax.axis_index("c")
    s = lax.axis_index("s")
    base = (c * 16 + s) * AGG_PS
    pltpu.sync_copy(zeros, acc.at[pl.ds(s * ROWS_PER_SUB, ROWS_PER_SUB)])
    plsc.subcore_barrier()

    ix = (ix0, ix1, ix2)
    dx = (dx0, dx1, dx2)
    rows = (rows0, rows1)
    isem = (is0, is1, is2)
    gsem = (gs0, gs1)
    ssem = (ss0, ss1)

    def islice(j):
        return pl.ds(base + j * AGG_CHUNK, AGG_CHUNK)

    def idx_start(k, j):
        pltpu.async_copy(src.at[islice(j)], ix[k], isem[k])
        pltpu.async_copy(dst.at[islice(j)], dx[k], isem[k])

    def idx_wait(k, j):
        pltpu.make_async_copy(src.at[islice(j)], ix[k], isem[k]).wait()
        pltpu.make_async_copy(dst.at[islice(j)], dx[k], isem[k]).wait()

    def g_start(r, k):
        pltpu.async_copy(h.at[ix[k]], rows[r], gsem[r])

    def g_wait(r, k):
        pltpu.make_async_copy(h.at[ix[k]], rows[r], gsem[r]).wait()

    def s_start(r, k):
        pltpu.async_copy(rows[r], acc.at[dx[k]], ssem[r], add=True)

    def s_wait(r, k):
        pltpu.make_async_copy(rows[r], acc.at[dx[k]], ssem[r]).wait()

    idx_start(0, 0)
    idx_wait(0, 0)
    g_start(0, 0)
    idx_start(1, 1)

    n_it = AGG_NCH // 6

    def body(i, carry):
        for b in range(6):
            j = 6 * i + b
            rs, rn = b % 2, 1 - b % 2
            kc = b % 3
            kn = (b + 1) % 3
            kp = (b + 2) % 3
            g_wait(rs, kc)
            s_start(rs, kc)
            if b == 0:
                @pl.when(i > 0)
                def _():
                    s_wait(rn, kp)
            else:
                s_wait(rn, kp)
            if b >= 4:
                @pl.when(i < n_it - 1)
                def _(j=j, kp=kp):
                    idx_start(kp, j + 2)
            else:
                idx_start(kp, j + 2)
            if b == 5:
                @pl.when(i < n_it - 1)
                def _(j=j, kn=kn, rn=rn):
                    idx_wait(kn, j + 1)
                    g_start(rn, kn)
            else:
                idx_wait(kn, j + 1)
                g_start(rn, kn)
        return carry
    lax.fori_loop(0, n_it, body, 0)
    s_wait(1, (AGG_NCH - 1) % 3)

    plsc.subcore_barrier()
    pltpu.sync_copy(acc.at[pl.ds(s * ROWS_PER_SUB, ROWS_PER_SUB)],
                    out.at[c, pl.ds(s * ROWS_PER_SUB, ROWS_PER_SUB)])


def _aggregate(h, src_p, dst_p):
    zeros = jnp.zeros((ROWS_PER_SUB, D), jnp.float32)
    fn = pl.kernel(
        _agg_body,
        out_type=jax.ShapeDtypeStruct((2, N_PAD, D), jnp.float32),
        mesh=_sc_mesh(),
        scratch_types=[
            pltpu.VMEM_SHARED((N_PAD, D), jnp.float32),
            pltpu.VMEM((AGG_CHUNK,), jnp.int32),
            pltpu.VMEM((AGG_CHUNK,), jnp.int32),
            pltpu.VMEM((AGG_CHUNK,), jnp.int32),
            pltpu.VMEM((AGG_CHUNK,), jnp.int32),
            pltpu.VMEM((AGG_CHUNK,), jnp.int32),
            pltpu.VMEM((AGG_CHUNK,), jnp.int32),
            pltpu.VMEM((AGG_CHUNK, D), jnp.float32),
            pltpu.VMEM((AGG_CHUNK, D), jnp.float32),
            pltpu.SemaphoreType.DMA,
            pltpu.SemaphoreType.DMA,
            pltpu.SemaphoreType.DMA,
            pltpu.SemaphoreType.DMA,
            pltpu.SemaphoreType.DMA,
            pltpu.SemaphoreType.DMA,
            pltpu.SemaphoreType.DMA,
        ],
    )
    return fn(h, src_p, dst_p, zeros)


def _prep_tc(deg_ref, f_ref, rs_ref, norm_ref, h0_0, h0_1, h0_2, h0_3):
    deg = deg_ref[...]
    norm = jnp.where(deg > 0.0, lax.rsqrt(jnp.maximum(deg, 1.0)), 0.0)
    norm_ref[...] = norm
    f = f_ref[...]
    for r, href in enumerate((h0_0, h0_1, h0_2, h0_3)):
        href[...] = f * rs_ref[r, :][None, :] * norm[:, 2 * r][:, None]


def _prep(degs, features, rs_pad):
    out_shape = (
        jax.ShapeDtypeStruct((N, 8), jnp.float32),
        *([jax.ShapeDtypeStruct((N, D), jnp.float32)] * 4),
    )
    return pl.pallas_call(
        _prep_tc,
        grid=(GRID_N,),
        in_specs=[
            pl.BlockSpec((NB, 8), lambda i: (i, 0)),
            pl.BlockSpec((NB, D), lambda i: (i, 0)),
            pl.BlockSpec((8, D), lambda i: (0, 0)),
        ],
        out_specs=(
            pl.BlockSpec((NB, 8), lambda i: (i, 0)),
            *([pl.BlockSpec((NB, D), lambda i: (i, 0))] * 4),
        ),
        out_shape=out_shape,
    )(degs, features, rs_pad)


def _dense_tc_stats(p_ref, nd_ref, w_ref, b_ref, y_ref, st_ref):
    agg = (p_ref[0] + p_ref[1]) * nd_ref[...]
    y = jnp.dot(agg, w_ref[...], preferred_element_type=jnp.float32) \
        + b_ref[0, :][None, :]
    y_ref[...] = y

    @pl.when(pl.program_id(0) == 0)
    def _():
        st_ref[...] = jnp.zeros_like(st_ref)
    st_ref[0:1, :] += jnp.sum(y, axis=0, keepdims=True)
    st_ref[1:2, :] += jnp.sum(y * y, axis=0, keepdims=True)


def _dense_tc(p_ref, nd_ref, w_ref, b_ref, y_ref):
    agg = (p_ref[0] + p_ref[1]) * nd_ref[...]
    y_ref[...] = jnp.dot(agg, w_ref[...], preferred_element_type=jnp.float32) \
        + b_ref[0, :][None, :]


def _dense(parts, norm_dst, w, b, with_stats):
    in_specs = [
        pl.BlockSpec((2, NB, D), lambda i: (0, i, 0)),
        pl.BlockSpec((NB, 1), lambda i: (i, 0)),
        pl.BlockSpec((D, D), lambda i: (0, 0)),
        pl.BlockSpec((1, D), lambda i: (0, 0)),
    ]
    if with_stats:
        return pl.pallas_call(
            _dense_tc_stats,
            grid=(GRID_N,),
            in_specs=in_specs,
            out_specs=(
                pl.BlockSpec((NB, D), lambda i: (i, 0)),
                pl.BlockSpec((8, D), lambda i: (0, 0)),
            ),
            out_shape=(
                jax.ShapeDtypeStruct((N, D), jnp.float32),
                jax.ShapeDtypeStruct((8, D), jnp.float32),
            ),
        )(parts, norm_dst, w, b[None, :])
    return pl.pallas_call(
        _dense_tc,
        grid=(GRID_N,),
        in_specs=in_specs,
        out_specs=pl.BlockSpec((NB, D), lambda i: (i, 0)),
        out_shape=jax.ShapeDtypeStruct((N, D), jnp.float32),
    )(parts, norm_dst, w, b[None, :])


def _post_tc(y_ref, st_ref, f_ref, r1_ref, ns_ref, g_ref, bt_ref, h1_ref):
    mean = st_ref[0:1, :] / N
    var = st_ref[1:2, :] / N - mean * mean
    z = (y_ref[...] - mean) * lax.rsqrt(var + 1e-5) * g_ref[0:1, :] \
        + bt_ref[0:1, :]
    z = jnp.where(z >= 0.0, z, NEG_SLOPE * z)
    emb = f_ref[...] + z
    h1_ref[...] = emb * r1_ref[0:1, :] * ns_ref[...]


def _post(y, stats, features, r1_row, norm_src, gamma, beta):
    return pl.pallas_call(
        _post_tc,
        grid=(GRID_N,),
        in_specs=[
            pl.BlockSpec((NB, D), lambda i: (i, 0)),
            pl.BlockSpec((8, D), lambda i: (0, 0)),
            pl.BlockSpec((NB, D), lambda i: (i, 0)),
            pl.BlockSpec((1, D), lambda i: (0, 0)),
            pl.BlockSpec((NB, 1), lambda i: (i, 0)),
            pl.BlockSpec((1, D), lambda i: (0, 0)),
            pl.BlockSpec((1, D), lambda i: (0, 0)),
        ],
        out_specs=pl.BlockSpec((NB, D), lambda i: (i, 0)),
        out_shape=jax.ShapeDtypeStruct((N, D), jnp.float32),
    )(y, stats, features, r1_row, norm_src, gamma[None, :], beta[None, :])


def _relvec_tc(rs_ref, w0_ref, b0_ref, w1_ref, b1_ref, r1_ref, r2_ref):
    r1 = jnp.dot(rs_ref[...], w0_ref[...], preferred_element_type=jnp.float32) \
        + b0_ref[0:1, :]
    r2 = jnp.dot(r1, w1_ref[...], preferred_element_type=jnp.float32) \
        + b1_ref[0:1, :]
    r1_ref[...] = r1
    r2_ref[...] = r2


def _relvec(rs_pad, w0, b0, w1, b1):
    return pl.pallas_call(
        _relvec_tc,
        out_shape=(
            jax.ShapeDtypeStruct((8, D), jnp.float32),
            jax.ShapeDtypeStruct((8, D), jnp.float32),
        ),
    )(rs_pad, w0, b0[None, :], w1, b1[None, :])


def kernel(features, poi_r, s_r, d_r, n_r,
           poi_edge_index, s_edge_index, d_edge_index, n_edge_index,
           W_gcn0, b_gcn0, W_gcn1, b_gcn1,
           bn_gamma0, bn_beta0,
           W_rel0, b_rel0, W_rel1, b_rel1):
    edges = [n_edge_index, poi_edge_index, s_edge_index, d_edge_index]
    srcs = [e[0] for e in edges]
    dsts = [e[1] for e in edges]
    n_pad_e = AGG_PS - E // 32
    pad_pos = (jnp.arange(32)[:, None] * n_pad_e + jnp.arange(n_pad_e)[None, :])
    pad_src = (pad_pos % N).astype(jnp.int32)
    pad_dst = (N + pad_pos % (N_PAD - N)).astype(jnp.int32)

    def _pad_idx(x, pad_block):
        return jnp.concatenate(
            [x.reshape(32, E // 32), pad_block], axis=1).reshape(-1)
    srcs_p = [_pad_idx(x, pad_src) for x in srcs]
    dsts_p = [_pad_idx(x, pad_dst) for x in dsts]
    rs_pad = jnp.concatenate(
        [jnp.stack([n_r, poi_r, s_r, d_r]), jnp.zeros((4, D), jnp.float32)], axis=0)

    deg_flat = _degrees(srcs, dsts)
    deg8 = _degred(deg_flat)
    degs = deg8[:, :N].transpose(1, 0)

    r1_pad, r2_pad = _relvec(rs_pad, W_rel0, b_rel0, W_rel1, b_rel1)
    norms, h0_0, h0_1, h0_2, h0_3 = _prep(degs, features, rs_pad)
    h0s = (h0_0, h0_1, h0_2, h0_3)

    embs = []
    for r in range(4):
        nd = norms[:, 2 * r + 1:2 * r + 2]
        ns = norms[:, 2 * r:2 * r + 1]
        parts0 = _aggregate(h0s[r], srcs_p[r], dsts_p[r])
        y, stats = _dense(parts0, nd, W_gcn0, b_gcn0, True)
        h1 = _post(y, stats, features, r1_pad[r:r + 1, :], ns,
                   bn_gamma0, bn_beta0)
        parts1 = _aggregate(h1, srcs_p[r], dsts_p[r])
        embs.append(_dense(parts1, nd, W_gcn1, b_gcn1, False))

    return (embs[0], embs[1], embs[2], embs[3],
            r2_pad[0], r2_pad[1], r2_pad[2], r2_pad[3])


# speedup vs baseline: 1.7896x; 1.1900x over previous
import functools

import jax
import jax.numpy as jnp
from jax import lax
from jax.experimental import pallas as pl
from jax.experimental.pallas import tpu as pltpu
from jax.experimental.pallas import tpu_sc as plsc

N = 10000
E = 320000
D = 128
N_PAD = 10240
ROWS_PER_SUB = 640
CHUNK = 80
NEG_SLOPE = 0.01
DEG_W = 16
NB = 1000
GRID_N = N // NB


def _sc_mesh():
    return plsc.VectorSubcoreMesh(core_axis_name="c", subcore_axis_name="s")


def _deg_body(e0s, e0d, e1s, e1d, e2s, e2d, e3s, e3d, out,
              cnt0, cnt1, cnt2, cnt3, cnt4, cnt5, cnt6, cnt7, idx_v, sem):
    del sem
    c = lax.axis_index("c")
    s = lax.axis_index("s")
    wid = c * 16 + s
    cnts = (cnt0, cnt1, cnt2, cnt3, cnt4, cnt5, cnt6, cnt7)
    zeros16 = jnp.zeros((16,), jnp.float32)

    def zbody(i, carry):
        for cnt in cnts:
            cnt[pl.ds(i * 16, 16)] = zeros16
        return carry
    lax.fori_loop(0, N_PAD // 16, zbody, 0)

    ones16 = jnp.full((16,), 1.0, jnp.float32)
    streams = (e0s, e0d, e1s, e1d, e2s, e2d, e3s, e3d)
    per_sub = E // 32

    for k in range(8):
        pltpu.sync_copy(streams[k].at[pl.ds(wid * per_sub, per_sub)], idx_v)

        def body(i, carry, k=k):
            idx16 = idx_v[pl.ds(i * 16, 16)]
            plsc.addupdate_scatter(cnts[k], [idx16], ones16)
            return carry
        lax.fori_loop(0, per_sub // 16, body, 0)

    for k in range(8):
        pltpu.sync_copy(cnts[k], out.at[pl.ds((wid * 8 + k) * N_PAD, N_PAD)])


def _degrees(srcs, dsts):
    fn = pl.kernel(
        _deg_body,
        out_type=jax.ShapeDtypeStruct((32 * 8 * N_PAD,), jnp.float32),
        mesh=_sc_mesh(),
        compiler_params=pltpu.CompilerParams(needs_layout_passes=False),
        scratch_types=[
            *([pltpu.VMEM((N_PAD,), jnp.float32)] * 8),
            pltpu.VMEM((E // 32,), jnp.int32),
            pltpu.SemaphoreType.DMA,
        ],
    )
    return fn(srcs[0], dsts[0], srcs[1], dsts[1], srcs[2], dsts[2],
              srcs[3], dsts[3])


def _degred_tc(x_ref, o_ref):
    acc = x_ref[0:8, :]
    for t in range(1, 32):
        acc = acc + x_ref[t * 8:(t + 1) * 8, :]
    o_ref[...] = acc


def _degred(deg_flat):
    x = deg_flat.reshape(32 * 8, N_PAD)
    return pl.pallas_call(
        _degred_tc,
        grid=(N_PAD // 128,),
        in_specs=[pl.BlockSpec((32 * 8, 128), lambda i: (0, i))],
        out_specs=pl.BlockSpec((8, 128), lambda i: (0, i)),
        out_shape=jax.ShapeDtypeStruct((8, N_PAD), jnp.float32),
    )(x)


AGG_CHUNK = 112
AGG_NCH = -(-(E // 32) // AGG_CHUNK)
AGG_PS = (AGG_NCH + 1) * AGG_CHUNK


def _agg_body(h, src, dst, zeros, out, acc,
              ix0, ix1, dx0, dx1, rows0, rows1, is0, is1, gs0, gs1):
    c = lax.axis_index("c")
    s = lax.axis_index("s")
    base = (c * 16 + s) * AGG_PS
    pltpu.sync_copy(zeros, acc.at[pl.ds(s * ROWS_PER_SUB, ROWS_PER_SUB)])
    plsc.subcore_barrier()

    ix = (ix0, ix1)
    dx = (dx0, dx1)
    rows = (rows0, rows1)
    isem = (is0, is1)
    gsem = (gs0, gs1)

    def islice(j):
        return pl.ds(base + j * AGG_CHUNK, AGG_CHUNK)

    def idx_start(slot, j):
        pltpu.async_copy(src.at[islice(j)], ix[slot], isem[slot])
        pltpu.async_copy(dst.at[islice(j)], dx[slot], isem[slot])

    def idx_wait(slot, j):
        pltpu.make_async_copy(src.at[islice(j)], ix[slot], isem[slot]).wait()
        pltpu.make_async_copy(dst.at[islice(j)], dx[slot], isem[slot]).wait()

    def g_start(slot):
        pltpu.async_copy(h.at[ix[slot]], rows[slot], gsem[slot])

    def fin(slot):
        pltpu.make_async_copy(h.at[ix[slot]], rows[slot], gsem[slot]).wait()
        pltpu.sync_copy(rows[slot], acc.at[dx[slot]], add=True)

    idx_start(0, 0)
    idx_wait(0, 0)
    g_start(0)
    idx_start(1, 1)

    def phase(j, slot):
        nxt = 1 - slot
        idx_wait(nxt, j + 1)
        g_start(nxt)
        fin(slot)
        idx_start(slot, j + 2)

    def body(i, carry):
        phase(2 * i, 0)
        phase(2 * i + 1, 1)
        return carry
    lax.fori_loop(0, (AGG_NCH - 1) // 2, body, 0)
    if AGG_NCH % 2 == 0:
        phase(AGG_NCH - 2, 0)
        fin(1)
        idx_wait(0, AGG_NCH)
    else:
        fin(0)
        idx_wait(1, AGG_NCH)

    plsc.subcore_barrier()
    pltpu.sync_copy(acc.at[pl.ds(s * ROWS_PER_SUB, ROWS_PER_SUB)],
                    out.at[c, pl.ds(s * ROWS_PER_SUB, ROWS_PER_SUB)])


def _aggregate(h, src_p, dst_p):
    zeros = jnp.zeros((ROWS_PER_SUB, D), jnp.float32)
    fn = pl.kernel(
        _agg_body,
        out_type=jax.ShapeDtypeStruct((2, N_PAD, D), jnp.float32),
        mesh=_sc_mesh(),
        scratch_types=[
            pltpu.VMEM_SHARED((N_PAD, D), jnp.float32),
            pltpu.VMEM((AGG_CHUNK,), jnp.int32),
            pltpu.VMEM((AGG_CHUNK,), jnp.int32),
            pltpu.VMEM((AGG_CHUNK,), jnp.int32),
            pltpu.VMEM((AGG_CHUNK,), jnp.int32),
            pltpu.VMEM((AGG_CHUNK, D), jnp.float32),
            pltpu.VMEM((AGG_CHUNK, D), jnp.float32),
            pltpu.SemaphoreType.DMA,
            pltpu.SemaphoreType.DMA,
            pltpu.SemaphoreType.DMA,
            pltpu.SemaphoreType.DMA,
        ],
    )
    return fn(h, src_p, dst_p, zeros)


def _prep_tc(deg_ref, f_ref, rs_ref, norm_ref, h0_0, h0_1, h0_2, h0_3):
    deg = deg_ref[...]
    norm = jnp.where(deg > 0.0, lax.rsqrt(jnp.maximum(deg, 1.0)), 0.0)
    norm_ref[...] = norm
    f = f_ref[...]
    for r, href in enumerate((h0_0, h0_1, h0_2, h0_3)):
        href[...] = f * rs_ref[r, :][None, :] * norm[:, 2 * r][:, None]


def _prep(degs, features, rs_pad):
    out_shape = (
        jax.ShapeDtypeStruct((N, 8), jnp.float32),
        *([jax.ShapeDtypeStruct((N, D), jnp.float32)] * 4),
    )
    return pl.pallas_call(
        _prep_tc,
        grid=(GRID_N,),
        in_specs=[
            pl.BlockSpec((NB, 8), lambda i: (i, 0)),
            pl.BlockSpec((NB, D), lambda i: (i, 0)),
            pl.BlockSpec((8, D), lambda i: (0, 0)),
        ],
        out_specs=(
            pl.BlockSpec((NB, 8), lambda i: (i, 0)),
            *([pl.BlockSpec((NB, D), lambda i: (i, 0))] * 4),
        ),
        out_shape=out_shape,
    )(degs, features, rs_pad)


def _dense_tc_stats(p_ref, nd_ref, w_ref, b_ref, y_ref, st_ref):
    agg = (p_ref[0] + p_ref[1]) * nd_ref[...]
    y = jnp.dot(agg, w_ref[...], preferred_element_type=jnp.float32) \
        + b_ref[0, :][None, :]
    y_ref[...] = y

    @pl.when(pl.program_id(0) == 0)
    def _():
        st_ref[...] = jnp.zeros_like(st_ref)
    st_ref[0:1, :] += jnp.sum(y, axis=0, keepdims=True)
    st_ref[1:2, :] += jnp.sum(y * y, axis=0, keepdims=True)


def _dense_tc(p_ref, nd_ref, w_ref, b_ref, y_ref):
    agg = (p_ref[0] + p_ref[1]) * nd_ref[...]
    y_ref[...] = jnp.dot(agg, w_ref[...], preferred_element_type=jnp.float32) \
        + b_ref[0, :][None, :]


def _dense(parts, norm_dst, w, b, with_stats):
    in_specs = [
        pl.BlockSpec((2, NB, D), lambda i: (0, i, 0)),
        pl.BlockSpec((NB, 1), lambda i: (i, 0)),
        pl.BlockSpec((D, D), lambda i: (0, 0)),
        pl.BlockSpec((1, D), lambda i: (0, 0)),
    ]
    if with_stats:
        return pl.pallas_call(
            _dense_tc_stats,
            grid=(GRID_N,),
            in_specs=in_specs,
            out_specs=(
                pl.BlockSpec((NB, D), lambda i: (i, 0)),
                pl.BlockSpec((8, D), lambda i: (0, 0)),
            ),
            out_shape=(
                jax.ShapeDtypeStruct((N, D), jnp.float32),
                jax.ShapeDtypeStruct((8, D), jnp.float32),
            ),
        )(parts, norm_dst, w, b[None, :])
    return pl.pallas_call(
        _dense_tc,
        grid=(GRID_N,),
        in_specs=in_specs,
        out_specs=pl.BlockSpec((NB, D), lambda i: (i, 0)),
        out_shape=jax.ShapeDtypeStruct((N, D), jnp.float32),
    )(parts, norm_dst, w, b[None, :])


def _post_tc(y_ref, st_ref, f_ref, r1_ref, ns_ref, g_ref, bt_ref, h1_ref):
    mean = st_ref[0:1, :] / N
    var = st_ref[1:2, :] / N - mean * mean
    z = (y_ref[...] - mean) * lax.rsqrt(var + 1e-5) * g_ref[0:1, :] \
        + bt_ref[0:1, :]
    z = jnp.where(z >= 0.0, z, NEG_SLOPE * z)
    emb = f_ref[...] + z
    h1_ref[...] = emb * r1_ref[0:1, :] * ns_ref[...]


def _post(y, stats, features, r1_row, norm_src, gamma, beta):
    return pl.pallas_call(
        _post_tc,
        grid=(GRID_N,),
        in_specs=[
            pl.BlockSpec((NB, D), lambda i: (i, 0)),
            pl.BlockSpec((8, D), lambda i: (0, 0)),
            pl.BlockSpec((NB, D), lambda i: (i, 0)),
            pl.BlockSpec((1, D), lambda i: (0, 0)),
            pl.BlockSpec((NB, 1), lambda i: (i, 0)),
            pl.BlockSpec((1, D), lambda i: (0, 0)),
            pl.BlockSpec((1, D), lambda i: (0, 0)),
        ],
        out_specs=pl.BlockSpec((NB, D), lambda i: (i, 0)),
        out_shape=jax.ShapeDtypeStruct((N, D), jnp.float32),
    )(y, stats, features, r1_row, norm_src, gamma[None, :], beta[None, :])


def _relvec_tc(rs_ref, w0_ref, b0_ref, w1_ref, b1_ref, r1_ref, r2_ref):
    r1 = jnp.dot(rs_ref[...], w0_ref[...], preferred_element_type=jnp.float32) \
        + b0_ref[0:1, :]
    r2 = jnp.dot(r1, w1_ref[...], preferred_element_type=jnp.float32) \
        + b1_ref[0:1, :]
    r1_ref[...] = r1
    r2_ref[...] = r2


def _relvec(rs_pad, w0, b0, w1, b1):
    return pl.pallas_call(
        _relvec_tc,
        out_shape=(
            jax.ShapeDtypeStruct((8, D), jnp.float32),
            jax.ShapeDtypeStruct((8, D), jnp.float32),
        ),
    )(rs_pad, w0, b0[None, :], w1, b1[None, :])


def kernel(features, poi_r, s_r, d_r, n_r,
           poi_edge_index, s_edge_index, d_edge_index, n_edge_index,
           W_gcn0, b_gcn0, W_gcn1, b_gcn1,
           bn_gamma0, bn_beta0,
           W_rel0, b_rel0, W_rel1, b_rel1):
    edges = [n_edge_index, poi_edge_index, s_edge_index, d_edge_index]
    srcs = [e[0] for e in edges]
    dsts = [e[1] for e in edges]
    n_pad_e = AGG_PS - E // 32
    pad_pos = (jnp.arange(32)[:, None] * n_pad_e + jnp.arange(n_pad_e)[None, :])
    pad_src = (pad_pos % N).astype(jnp.int32)
    pad_dst = (N + pad_pos % (N_PAD - N)).astype(jnp.int32)

    def _pad_idx(x, pad_block):
        return jnp.concatenate(
            [x.reshape(32, E // 32), pad_block], axis=1).reshape(-1)
    srcs_p = [_pad_idx(x, pad_src) for x in srcs]
    dsts_p = [_pad_idx(x, pad_dst) for x in dsts]
    rs_pad = jnp.concatenate(
        [jnp.stack([n_r, poi_r, s_r, d_r]), jnp.zeros((4, D), jnp.float32)], axis=0)

    deg_flat = _degrees(srcs, dsts)
    deg8 = _degred(deg_flat)
    degs = deg8[:, :N].transpose(1, 0)

    r1_pad, r2_pad = _relvec(rs_pad, W_rel0, b_rel0, W_rel1, b_rel1)
    norms, h0_0, h0_1, h0_2, h0_3 = _prep(degs, features, rs_pad)
    h0s = (h0_0, h0_1, h0_2, h0_3)

    embs = []
    for r in range(4):
        nd = norms[:, 2 * r + 1:2 * r + 2]
        ns = norms[:, 2 * r:2 * r + 1]
        parts0 = _aggregate(h0s[r], srcs_p[r], dsts_p[r])
        y, stats = _dense(parts0, nd, W_gcn0, b_gcn0, True)
        h1 = _post(y, stats, features, r1_pad[r:r + 1, :], ns,
                   bn_gamma0, bn_beta0)
        parts1 = _aggregate(h1, srcs_p[r], dsts_p[r])
        embs.append(_dense(parts1, nd, W_gcn1, b_gcn1, False))

    return (embs[0], embs[1], embs[2], embs[3],
            r2_pad[0], r2_pad[1], r2_pad[2], r2_pad[3])


# speedup vs baseline: 1.8233x; 1.0188x over previous
import functools

import jax
import jax.numpy as jnp
from jax import lax
from jax.experimental import pallas as pl
from jax.experimental.pallas import tpu as pltpu
from jax.experimental.pallas import tpu_sc as plsc

N = 10000
E = 320000
D = 128
N_PAD = 10240
ROWS_PER_SUB = 640
CHUNK = 80
NEG_SLOPE = 0.01
DEG_W = 16
NB = 1000
GRID_N = N // NB


def _sc_mesh():
    return plsc.VectorSubcoreMesh(core_axis_name="c", subcore_axis_name="s")


def _deg_body(e0s, e0d, e1s, e1d, e2s, e2d, e3s, e3d, out,
              cnt0, cnt1, cnt2, cnt3, cnt4, cnt5, cnt6, cnt7, idx_v, sem):
    del sem
    c = lax.axis_index("c")
    s = lax.axis_index("s")
    wid = c * 16 + s
    cnts = (cnt0, cnt1, cnt2, cnt3, cnt4, cnt5, cnt6, cnt7)
    zeros16 = jnp.zeros((16,), jnp.float32)

    def zbody(i, carry):
        for cnt in cnts:
            cnt[pl.ds(i * 16, 16)] = zeros16
        return carry
    lax.fori_loop(0, N_PAD // 16, zbody, 0)

    ones16 = jnp.full((16,), 1.0, jnp.float32)
    streams = (e0s, e0d, e1s, e1d, e2s, e2d, e3s, e3d)
    per_sub = E // 32

    for k in range(8):
        pltpu.sync_copy(streams[k].at[pl.ds(wid * per_sub, per_sub)], idx_v)

        def body(i, carry, k=k):
            idx16 = idx_v[pl.ds(i * 16, 16)]
            plsc.addupdate_scatter(cnts[k], [idx16], ones16)
            return carry
        lax.fori_loop(0, per_sub // 16, body, 0)

    for k in range(8):
        pltpu.sync_copy(cnts[k], out.at[pl.ds((wid * 8 + k) * N_PAD, N_PAD)])


def _degrees(srcs, dsts):
    fn = pl.kernel(
        _deg_body,
        out_type=jax.ShapeDtypeStruct((32 * 8 * N_PAD,), jnp.float32),
        mesh=_sc_mesh(),
        compiler_params=pltpu.CompilerParams(needs_layout_passes=False),
        scratch_types=[
            *([pltpu.VMEM((N_PAD,), jnp.float32)] * 8),
            pltpu.VMEM((E // 32,), jnp.int32),
            pltpu.SemaphoreType.DMA,
        ],
    )
    return fn(srcs[0], dsts[0], srcs[1], dsts[1], srcs[2], dsts[2],
              srcs[3], dsts[3])


def _degred_tc(x_ref, o_ref):
    acc = x_ref[0:8, :]
    for t in range(1, 32):
        acc = acc + x_ref[t * 8:(t + 1) * 8, :]
    o_ref[...] = acc


def _degred(deg_flat):
    x = deg_flat.reshape(32 * 8, N_PAD)
    return pl.pallas_call(
        _degred_tc,
        grid=(N_PAD // 128,),
        in_specs=[pl.BlockSpec((32 * 8, 128), lambda i: (0, i))],
        out_specs=pl.BlockSpec((8, 128), lambda i: (0, i)),
        out_shape=jax.ShapeDtypeStruct((8, N_PAD), jnp.float32),
    )(x)


AGG_CHUNK = 120
AGG_NCH = -(-(E // 32) // AGG_CHUNK)
AGG_PS = (AGG_NCH + 1) * AGG_CHUNK


def _agg_body(h, src, dst, zeros, out, acc,
              ix0, ix1, dx0, dx1, rows0, rows1, is0, is1, gs0, gs1):
    c = lax.axis_index("c")
    s = lax.axis_index("s")
    base = (c * 16 + s) * AGG_PS
    pltpu.sync_copy(zeros, acc.at[pl.ds(s * ROWS_PER_SUB, ROWS_PER_SUB)])
    plsc.subcore_barrier()

    ix = (ix0, ix1)
    dx = (dx0, dx1)
    rows = (rows0, rows1)
    isem = (is0, is1)
    gsem = (gs0, gs1)

    def islice(j):
        return pl.ds(base + j * AGG_CHUNK, AGG_CHUNK)

    def idx_start(slot, j):
        pltpu.async_copy(src.at[islice(j)], ix[slot], isem[slot])
        pltpu.async_copy(dst.at[islice(j)], dx[slot], isem[slot])

    def idx_wait(slot, j):
        pltpu.make_async_copy(src.at[islice(j)], ix[slot], isem[slot]).wait()
        pltpu.make_async_copy(dst.at[islice(j)], dx[slot], isem[slot]).wait()

    def g_start(slot):
        pltpu.async_copy(h.at[ix[slot]], rows[slot], gsem[slot])

    def fin(slot):
        pltpu.make_async_copy(h.at[ix[slot]], rows[slot], gsem[slot]).wait()
        pltpu.sync_copy(rows[slot], acc.at[dx[slot]], add=True)

    idx_start(0, 0)
    idx_wait(0, 0)
    g_start(0)
    idx_start(1, 1)

    def phase(j, slot):
        nxt = 1 - slot
        idx_wait(nxt, j + 1)
        g_start(nxt)
        fin(slot)
        idx_start(slot, j + 2)

    def body(i, carry):
        phase(2 * i, 0)
        phase(2 * i + 1, 1)
        return carry
    lax.fori_loop(0, (AGG_NCH - 1) // 2, body, 0)
    if AGG_NCH % 2 == 0:
        phase(AGG_NCH - 2, 0)
        fin(1)
        idx_wait(0, AGG_NCH)
    else:
        fin(0)
        idx_wait(1, AGG_NCH)

    plsc.subcore_barrier()
    pltpu.sync_copy(acc.at[pl.ds(s * ROWS_PER_SUB, ROWS_PER_SUB)],
                    out.at[c, pl.ds(s * ROWS_PER_SUB, ROWS_PER_SUB)])


def _aggregate(h, src_p, dst_p):
    zeros = jnp.zeros((ROWS_PER_SUB, D), jnp.float32)
    fn = pl.kernel(
        _agg_body,
        out_type=jax.ShapeDtypeStruct((2, N_PAD, D), jnp.float32),
        mesh=_sc_mesh(),
        scratch_types=[
            pltpu.VMEM_SHARED((N_PAD, D), jnp.float32),
            pltpu.VMEM((AGG_CHUNK,), jnp.int32),
            pltpu.VMEM((AGG_CHUNK,), jnp.int32),
            pltpu.VMEM((AGG_CHUNK,), jnp.int32),
            pltpu.VMEM((AGG_CHUNK,), jnp.int32),
            pltpu.VMEM((AGG_CHUNK, D), jnp.float32),
            pltpu.VMEM((AGG_CHUNK, D), jnp.float32),
            pltpu.SemaphoreType.DMA,
            pltpu.SemaphoreType.DMA,
            pltpu.SemaphoreType.DMA,
            pltpu.SemaphoreType.DMA,
        ],
    )
    return fn(h, src_p, dst_p, zeros)


def _prep_tc(deg_ref, f_ref, rs_ref, norm_ref, h0_0, h0_1, h0_2, h0_3):
    deg = deg_ref[...]
    norm = jnp.where(deg > 0.0, lax.rsqrt(jnp.maximum(deg, 1.0)), 0.0)
    norm_ref[...] = norm
    f = f_ref[...]
    for r, href in enumerate((h0_0, h0_1, h0_2, h0_3)):
        href[...] = f * rs_ref[r, :][None, :] * norm[:, 2 * r][:, None]


def _prep(degs, features, rs_pad):
    out_shape = (
        jax.ShapeDtypeStruct((N, 8), jnp.float32),
        *([jax.ShapeDtypeStruct((N, D), jnp.float32)] * 4),
    )
    return pl.pallas_call(
        _prep_tc,
        grid=(GRID_N,),
        in_specs=[
            pl.BlockSpec((NB, 8), lambda i: (i, 0)),
            pl.BlockSpec((NB, D), lambda i: (i, 0)),
            pl.BlockSpec((8, D), lambda i: (0, 0)),
        ],
        out_specs=(
            pl.BlockSpec((NB, 8), lambda i: (i, 0)),
            *([pl.BlockSpec((NB, D), lambda i: (i, 0))] * 4),
        ),
        out_shape=out_shape,
    )(degs, features, rs_pad)


def _dense_tc_stats(p_ref, nd_ref, w_ref, b_ref, y_ref, st_ref):
    agg = (p_ref[0] + p_ref[1]) * nd_ref[...]
    y = jnp.dot(agg, w_ref[...], preferred_element_type=jnp.float32) \
        + b_ref[0, :][None, :]
    y_ref[...] = y

    @pl.when(pl.program_id(0) == 0)
    def _():
        st_ref[...] = jnp.zeros_like(st_ref)
    st_ref[0:1, :] += jnp.sum(y, axis=0, keepdims=True)
    st_ref[1:2, :] += jnp.sum(y * y, axis=0, keepdims=True)


def _dense_tc(p_ref, nd_ref, w_ref, b_ref, y_ref):
    agg = (p_ref[0] + p_ref[1]) * nd_ref[...]
    y_ref[...] = jnp.dot(agg, w_ref[...], preferred_element_type=jnp.float32) \
        + b_ref[0, :][None, :]


def _dense(parts, norm_dst, w, b, with_stats):
    in_specs = [
        pl.BlockSpec((2, NB, D), lambda i: (0, i, 0)),
        pl.BlockSpec((NB, 1), lambda i: (i, 0)),
        pl.BlockSpec((D, D), lambda i: (0, 0)),
        pl.BlockSpec((1, D), lambda i: (0, 0)),
    ]
    if with_stats:
        return pl.pallas_call(
            _dense_tc_stats,
            grid=(GRID_N,),
            in_specs=in_specs,
            out_specs=(
                pl.BlockSpec((NB, D), lambda i: (i, 0)),
                pl.BlockSpec((8, D), lambda i: (0, 0)),
            ),
            out_shape=(
                jax.ShapeDtypeStruct((N, D), jnp.float32),
                jax.ShapeDtypeStruct((8, D), jnp.float32),
            ),
        )(parts, norm_dst, w, b[None, :])
    return pl.pallas_call(
        _dense_tc,
        grid=(GRID_N,),
        in_specs=in_specs,
        out_specs=pl.BlockSpec((NB, D), lambda i: (i, 0)),
        out_shape=jax.ShapeDtypeStruct((N, D), jnp.float32),
    )(parts, norm_dst, w, b[None, :])


def _post_tc(y_ref, st_ref, f_ref, r1_ref, ns_ref, g_ref, bt_ref, h1_ref):
    mean = st_ref[0:1, :] / N
    var = st_ref[1:2, :] / N - mean * mean
    z = (y_ref[...] - mean) * lax.rsqrt(var + 1e-5) * g_ref[0:1, :] \
        + bt_ref[0:1, :]
    z = jnp.where(z >= 0.0, z, NEG_SLOPE * z)
    emb = f_ref[...] + z
    h1_ref[...] = emb * r1_ref[0:1, :] * ns_ref[...]


def _post(y, stats, features, r1_row, norm_src, gamma, beta):
    return pl.pallas_call(
        _post_tc,
        grid=(GRID_N,),
        in_specs=[
            pl.BlockSpec((NB, D), lambda i: (i, 0)),
            pl.BlockSpec((8, D), lambda i: (0, 0)),
            pl.BlockSpec((NB, D), lambda i: (i, 0)),
            pl.BlockSpec((1, D), lambda i: (0, 0)),
            pl.BlockSpec((NB, 1), lambda i: (i, 0)),
            pl.BlockSpec((1, D), lambda i: (0, 0)),
            pl.BlockSpec((1, D), lambda i: (0, 0)),
        ],
        out_specs=pl.BlockSpec((NB, D), lambda i: (i, 0)),
        out_shape=jax.ShapeDtypeStruct((N, D), jnp.float32),
    )(y, stats, features, r1_row, norm_src, gamma[None, :], beta[None, :])


def _relvec_tc(rs_ref, w0_ref, b0_ref, w1_ref, b1_ref, r1_ref, r2_ref):
    r1 = jnp.dot(rs_ref[...], w0_ref[...], preferred_element_type=jnp.float32) \
        + b0_ref[0:1, :]
    r2 = jnp.dot(r1, w1_ref[...], preferred_element_type=jnp.float32) \
        + b1_ref[0:1, :]
    r1_ref[...] = r1
    r2_ref[...] = r2


def _relvec(rs_pad, w0, b0, w1, b1):
    return pl.pallas_call(
        _relvec_tc,
        out_shape=(
            jax.ShapeDtypeStruct((8, D), jnp.float32),
            jax.ShapeDtypeStruct((8, D), jnp.float32),
        ),
    )(rs_pad, w0, b0[None, :], w1, b1[None, :])


def kernel(features, poi_r, s_r, d_r, n_r,
           poi_edge_index, s_edge_index, d_edge_index, n_edge_index,
           W_gcn0, b_gcn0, W_gcn1, b_gcn1,
           bn_gamma0, bn_beta0,
           W_rel0, b_rel0, W_rel1, b_rel1):
    edges = [n_edge_index, poi_edge_index, s_edge_index, d_edge_index]
    srcs = [e[0] for e in edges]
    dsts = [e[1] for e in edges]
    n_pad_e = AGG_PS - E // 32
    pad_pos = (jnp.arange(32)[:, None] * n_pad_e + jnp.arange(n_pad_e)[None, :])
    pad_src = (pad_pos % N).astype(jnp.int32)
    pad_dst = (N + pad_pos % (N_PAD - N)).astype(jnp.int32)

    def _pad_idx(x, pad_block):
        return jnp.concatenate(
            [x.reshape(32, E // 32), pad_block], axis=1).reshape(-1)
    srcs_p = [_pad_idx(x, pad_src) for x in srcs]
    dsts_p = [_pad_idx(x, pad_dst) for x in dsts]
    rs_pad = jnp.concatenate(
        [jnp.stack([n_r, poi_r, s_r, d_r]), jnp.zeros((4, D), jnp.float32)], axis=0)

    deg_flat = _degrees(srcs, dsts)
    deg8 = _degred(deg_flat)
    degs = deg8[:, :N].transpose(1, 0)

    r1_pad, r2_pad = _relvec(rs_pad, W_rel0, b_rel0, W_rel1, b_rel1)
    norms, h0_0, h0_1, h0_2, h0_3 = _prep(degs, features, rs_pad)
    h0s = (h0_0, h0_1, h0_2, h0_3)

    embs = []
    for r in range(4):
        nd = norms[:, 2 * r + 1:2 * r + 2]
        ns = norms[:, 2 * r:2 * r + 1]
        parts0 = _aggregate(h0s[r], srcs_p[r], dsts_p[r])
        y, stats = _dense(parts0, nd, W_gcn0, b_gcn0, True)
        h1 = _post(y, stats, features, r1_pad[r:r + 1, :], ns,
                   bn_gamma0, bn_beta0)
        parts1 = _aggregate(h1, srcs_p[r], dsts_p[r])
        embs.append(_dense(parts1, nd, W_gcn1, b_gcn1, False))

    return (embs[0], embs[1], embs[2], embs[3],
            r2_pad[0], r2_pad[1], r2_pad[2], r2_pad[3])


# speedup vs baseline: 1.8655x; 1.0232x over previous
import functools

import jax
import jax.numpy as jnp
from jax import lax
from jax.experimental import pallas as pl
from jax.experimental.pallas import tpu as pltpu
from jax.experimental.pallas import tpu_sc as plsc

N = 10000
E = 320000
D = 128
N_PAD = 10240
ROWS_PER_SUB = 640
CHUNK = 80
NEG_SLOPE = 0.01
DEG_W = 16
NB = 1000
GRID_N = N // NB


def _sc_mesh():
    return plsc.VectorSubcoreMesh(core_axis_name="c", subcore_axis_name="s")


def _deg_body(e0s, e0d, e1s, e1d, e2s, e2d, e3s, e3d, out,
              cnt0, cnt1, cnt2, cnt3, cnt4, cnt5, cnt6, cnt7,
              idx_a, idx_b, sa, sb):
    c = lax.axis_index("c")
    s = lax.axis_index("s")
    wid = c * 16 + s
    cnts = (cnt0, cnt1, cnt2, cnt3, cnt4, cnt5, cnt6, cnt7)
    zeros16 = jnp.zeros((16,), jnp.float32)

    def zbody(i, carry):
        for cnt in cnts:
            cnt[pl.ds(i * 16, 16)] = zeros16
        return carry
    lax.fori_loop(0, N_PAD // 16, zbody, 0)

    ones16 = jnp.full((16,), 1.0, jnp.float32)
    streams = (e0s, e0d, e1s, e1d, e2s, e2d, e3s, e3d)
    per_sub = E // 32
    bufs = (idx_a, idx_b)
    sems = (sa, sb)

    def sl():
        return pl.ds(wid * per_sub, per_sub)

    pltpu.async_copy(streams[0].at[sl()], bufs[0], sems[0])
    for k in range(8):
        cur, csem = bufs[k % 2], sems[k % 2]
        pltpu.make_async_copy(streams[k].at[sl()], cur, csem).wait()
        if k < 7:
            pltpu.async_copy(streams[k + 1].at[sl()],
                             bufs[(k + 1) % 2], sems[(k + 1) % 2])

        def body(i, carry, k=k, cur=cur):
            idx16 = cur[pl.ds(i * 16, 16)]
            plsc.addupdate_scatter(cnts[k], [idx16], ones16)
            return carry
        lax.fori_loop(0, per_sub // 16, body, 0)

    for k in range(8):
        pltpu.sync_copy(cnts[k], out.at[pl.ds((wid * 8 + k) * N_PAD, N_PAD)])


def _degrees(srcs, dsts):
    fn = pl.kernel(
        _deg_body,
        out_type=jax.ShapeDtypeStruct((32 * 8 * N_PAD,), jnp.float32),
        mesh=_sc_mesh(),
        compiler_params=pltpu.CompilerParams(needs_layout_passes=False),
        scratch_types=[
            *([pltpu.VMEM((N_PAD,), jnp.float32)] * 8),
            pltpu.VMEM((E // 32,), jnp.int32),
            pltpu.VMEM((E // 32,), jnp.int32),
            pltpu.SemaphoreType.DMA,
            pltpu.SemaphoreType.DMA,
        ],
    )
    return fn(srcs[0], dsts[0], srcs[1], dsts[1], srcs[2], dsts[2],
              srcs[3], dsts[3])


def _degred_tc(x_ref, o_ref):
    acc = x_ref[0:8, :]
    for t in range(1, 32):
        acc = acc + x_ref[t * 8:(t + 1) * 8, :]
    o_ref[...] = acc


def _degred(deg_flat):
    x = deg_flat.reshape(32 * 8, N_PAD)
    return pl.pallas_call(
        _degred_tc,
        grid=(N_PAD // 128,),
        in_specs=[pl.BlockSpec((32 * 8, 128), lambda i: (0, i))],
        out_specs=pl.BlockSpec((8, 128), lambda i: (0, i)),
        out_shape=jax.ShapeDtypeStruct((8, N_PAD), jnp.float32),
    )(x)


AGG_CHUNK = 128
AGG_NCH = -(-(E // 32) // AGG_CHUNK)
AGG_PS = (AGG_NCH + 1) * AGG_CHUNK


def _agg_body(h, src, dst, zeros, out, acc,
              ix0, ix1, dx0, dx1, rows0, rows1, is0, is1, gs0, gs1):
    c = lax.axis_index("c")
    s = lax.axis_index("s")
    base = (c * 16 + s) * AGG_PS
    pltpu.sync_copy(zeros, acc.at[pl.ds(s * ROWS_PER_SUB, ROWS_PER_SUB)])
    plsc.subcore_barrier()

    ix = (ix0, ix1)
    dx = (dx0, dx1)
    rows = (rows0, rows1)
    isem = (is0, is1)
    gsem = (gs0, gs1)

    def islice(j):
        return pl.ds(base + j * AGG_CHUNK, AGG_CHUNK)

    def idx_start(slot, j):
        pltpu.async_copy(src.at[islice(j)], ix[slot], isem[slot])
        pltpu.async_copy(dst.at[islice(j)], dx[slot], isem[slot])

    def idx_wait(slot, j):
        pltpu.make_async_copy(src.at[islice(j)], ix[slot], isem[slot]).wait()
        pltpu.make_async_copy(dst.at[islice(j)], dx[slot], isem[slot]).wait()

    def g_start(slot):
        pltpu.async_copy(h.at[ix[slot]], rows[slot], gsem[slot])

    def fin(slot):
        pltpu.make_async_copy(h.at[ix[slot]], rows[slot], gsem[slot]).wait()
        pltpu.sync_copy(rows[slot], acc.at[dx[slot]], add=True)

    idx_start(0, 0)
    idx_wait(0, 0)
    g_start(0)
    idx_start(1, 1)

    def phase(j, slot):
        nxt = 1 - slot
        idx_wait(nxt, j + 1)
        g_start(nxt)
        fin(slot)
        idx_start(slot, j + 2)

    def body(i, carry):
        phase(2 * i, 0)
        phase(2 * i + 1, 1)
        return carry
    lax.fori_loop(0, (AGG_NCH - 1) // 2, body, 0)
    if AGG_NCH % 2 == 0:
        phase(AGG_NCH - 2, 0)
        fin(1)
        idx_wait(0, AGG_NCH)
    else:
        fin(0)
        idx_wait(1, AGG_NCH)

    plsc.subcore_barrier()
    pltpu.sync_copy(acc.at[pl.ds(s * ROWS_PER_SUB, ROWS_PER_SUB)],
                    out.at[c, pl.ds(s * ROWS_PER_SUB, ROWS_PER_SUB)])


def _aggregate(h, src_p, dst_p):
    zeros = jnp.zeros((ROWS_PER_SUB, D), jnp.float32)
    fn = pl.kernel(
        _agg_body,
        out_type=jax.ShapeDtypeStruct((2, N_PAD, D), jnp.float32),
        mesh=_sc_mesh(),
        scratch_types=[
            pltpu.VMEM_SHARED((N_PAD, D), jnp.float32),
            pltpu.VMEM((AGG_CHUNK,), jnp.int32),
            pltpu.VMEM((AGG_CHUNK,), jnp.int32),
            pltpu.VMEM((AGG_CHUNK,), jnp.int32),
            pltpu.VMEM((AGG_CHUNK,), jnp.int32),
            pltpu.VMEM((AGG_CHUNK, D), jnp.float32),
            pltpu.VMEM((AGG_CHUNK, D), jnp.float32),
            pltpu.SemaphoreType.DMA,
            pltpu.SemaphoreType.DMA,
            pltpu.SemaphoreType.DMA,
            pltpu.SemaphoreType.DMA,
        ],
    )
    return fn(h, src_p, dst_p, zeros)


def _prep_tc(deg_ref, f_ref, rs_ref, norm_ref, h0_0, h0_1, h0_2, h0_3):
    deg = deg_ref[...]
    norm = jnp.where(deg > 0.0, lax.rsqrt(jnp.maximum(deg, 1.0)), 0.0)
    norm_ref[...] = norm
    f = f_ref[...]
    for r, href in enumerate((h0_0, h0_1, h0_2, h0_3)):
        href[...] = f * rs_ref[r, :][None, :] * norm[:, 2 * r][:, None]


def _prep(degs, features, rs_pad):
    out_shape = (
        jax.ShapeDtypeStruct((N, 8), jnp.float32),
        *([jax.ShapeDtypeStruct((N, D), jnp.float32)] * 4),
    )
    return pl.pallas_call(
        _prep_tc,
        grid=(GRID_N,),
        in_specs=[
            pl.BlockSpec((NB, 8), lambda i: (i, 0)),
            pl.BlockSpec((NB, D), lambda i: (i, 0)),
            pl.BlockSpec((8, D), lambda i: (0, 0)),
        ],
        out_specs=(
            pl.BlockSpec((NB, 8), lambda i: (i, 0)),
            *([pl.BlockSpec((NB, D), lambda i: (i, 0))] * 4),
        ),
        out_shape=out_shape,
    )(degs, features, rs_pad)


def _dense_tc_stats(p_ref, nd_ref, w_ref, b_ref, y_ref, st_ref):
    agg = (p_ref[0] + p_ref[1]) * nd_ref[...]
    y = jnp.dot(agg, w_ref[...], preferred_element_type=jnp.float32) \
        + b_ref[0, :][None, :]
    y_ref[...] = y

    @pl.when(pl.program_id(0) == 0)
    def _():
        st_ref[...] = jnp.zeros_like(st_ref)
    st_ref[0:1, :] += jnp.sum(y, axis=0, keepdims=True)
    st_ref[1:2, :] += jnp.sum(y * y, axis=0, keepdims=True)


def _dense_tc(p_ref, nd_ref, w_ref, b_ref, y_ref):
    agg = (p_ref[0] + p_ref[1]) * nd_ref[...]
    y_ref[...] = jnp.dot(agg, w_ref[...], preferred_element_type=jnp.float32) \
        + b_ref[0, :][None, :]


def _dense(parts, norm_dst, w, b, with_stats):
    in_specs = [
        pl.BlockSpec((2, NB, D), lambda i: (0, i, 0)),
        pl.BlockSpec((NB, 1), lambda i: (i, 0)),
        pl.BlockSpec((D, D), lambda i: (0, 0)),
        pl.BlockSpec((1, D), lambda i: (0, 0)),
    ]
    if with_stats:
        return pl.pallas_call(
            _dense_tc_stats,
            grid=(GRID_N,),
            in_specs=in_specs,
            out_specs=(
                pl.BlockSpec((NB, D), lambda i: (i, 0)),
                pl.BlockSpec((8, D), lambda i: (0, 0)),
            ),
            out_shape=(
                jax.ShapeDtypeStruct((N, D), jnp.float32),
                jax.ShapeDtypeStruct((8, D), jnp.float32),
            ),
        )(parts, norm_dst, w, b[None, :])
    return pl.pallas_call(
        _dense_tc,
        grid=(GRID_N,),
        in_specs=in_specs,
        out_specs=pl.BlockSpec((NB, D), lambda i: (i, 0)),
        out_shape=jax.ShapeDtypeStruct((N, D), jnp.float32),
    )(parts, norm_dst, w, b[None, :])


def _post_tc(y_ref, st_ref, f_ref, r1_ref, ns_ref, g_ref, bt_ref, h1_ref):
    mean = st_ref[0:1, :] / N
    var = st_ref[1:2, :] / N - mean * mean
    z = (y_ref[...] - mean) * lax.rsqrt(var + 1e-5) * g_ref[0:1, :] \
        + bt_ref[0:1, :]
    z = jnp.where(z >= 0.0, z, NEG_SLOPE * z)
    emb = f_ref[...] + z
    h1_ref[...] = emb * r1_ref[0:1, :] * ns_ref[...]


def _post(y, stats, features, r1_row, norm_src, gamma, beta):
    return pl.pallas_call(
        _post_tc,
        grid=(GRID_N,),
        in_specs=[
            pl.BlockSpec((NB, D), lambda i: (i, 0)),
            pl.BlockSpec((8, D), lambda i: (0, 0)),
            pl.BlockSpec((NB, D), lambda i: (i, 0)),
            pl.BlockSpec((1, D), lambda i: (0, 0)),
            pl.BlockSpec((NB, 1), lambda i: (i, 0)),
            pl.BlockSpec((1, D), lambda i: (0, 0)),
            pl.BlockSpec((1, D), lambda i: (0, 0)),
        ],
        out_specs=pl.BlockSpec((NB, D), lambda i: (i, 0)),
        out_shape=jax.ShapeDtypeStruct((N, D), jnp.float32),
    )(y, stats, features, r1_row, norm_src, gamma[None, :], beta[None, :])


def _relvec_tc(rs_ref, w0_ref, b0_ref, w1_ref, b1_ref, r1_ref, r2_ref):
    r1 = jnp.dot(rs_ref[...], w0_ref[...], preferred_element_type=jnp.float32) \
        + b0_ref[0:1, :]
    r2 = jnp.dot(r1, w1_ref[...], preferred_element_type=jnp.float32) \
        + b1_ref[0:1, :]
    r1_ref[...] = r1
    r2_ref[...] = r2


def _relvec(rs_pad, w0, b0, w1, b1):
    return pl.pallas_call(
        _relvec_tc,
        out_shape=(
            jax.ShapeDtypeStruct((8, D), jnp.float32),
            jax.ShapeDtypeStruct((8, D), jnp.float32),
        ),
    )(rs_pad, w0, b0[None, :], w1, b1[None, :])


def kernel(features, poi_r, s_r, d_r, n_r,
           poi_edge_index, s_edge_index, d_edge_index, n_edge_index,
           W_gcn0, b_gcn0, W_gcn1, b_gcn1,
           bn_gamma0, bn_beta0,
           W_rel0, b_rel0, W_rel1, b_rel1):
    edges = [n_edge_index, poi_edge_index, s_edge_index, d_edge_index]
    srcs = [e[0] for e in edges]
    dsts = [e[1] for e in edges]
    n_pad_e = AGG_PS - E // 32
    pad_pos = (jnp.arange(32)[:, None] * n_pad_e + jnp.arange(n_pad_e)[None, :])
    pad_src = (pad_pos % N).astype(jnp.int32)
    pad_dst = (N + pad_pos % (N_PAD - N)).astype(jnp.int32)

    def _pad_idx(x, pad_block):
        return jnp.concatenate(
            [x.reshape(32, E // 32), pad_block], axis=1).reshape(-1)
    srcs_p = [_pad_idx(x, pad_src) for x in srcs]
    dsts_p = [_pad_idx(x, pad_dst) for x in dsts]
    rs_pad = jnp.concatenate(
        [jnp.stack([n_r, poi_r, s_r, d_r]), jnp.zeros((4, D), jnp.float32)], axis=0)

    deg_flat = _degrees(srcs, dsts)
    deg8 = _degred(deg_flat)
    degs = deg8[:, :N].transpose(1, 0)

    r1_pad, r2_pad = _relvec(rs_pad, W_rel0, b_rel0, W_rel1, b_rel1)
    norms, h0_0, h0_1, h0_2, h0_3 = _prep(degs, features, rs_pad)
    h0s = (h0_0, h0_1, h0_2, h0_3)

    embs = []
    for r in range(4):
        nd = norms[:, 2 * r + 1:2 * r + 2]
        ns = norms[:, 2 * r:2 * r + 1]
        parts0 = _aggregate(h0s[r], srcs_p[r], dsts_p[r])
        y, stats = _dense(parts0, nd, W_gcn0, b_gcn0, True)
        h1 = _post(y, stats, features, r1_pad[r:r + 1, :], ns,
                   bn_gamma0, bn_beta0)
        parts1 = _aggregate(h1, srcs_p[r], dsts_p[r])
        embs.append(_dense(parts1, nd, W_gcn1, b_gcn1, False))

    return (embs[0], embs[1], embs[2], embs[3],
            r2_pad[0], r2_pad[1], r2_pad[2], r2_pad[3])


# speedup vs baseline: 1.8657x; 1.0001x over previous
import jax
import jax.numpy as jnp
from jax import lax
from jax.experimental import pallas as pl
from jax.experimental.pallas import tpu as pltpu
from jax.experimental.pallas import tpu_sc as plsc

N = 10000
E = 320000
D = 128
N_PAD = 10240
ROWS_PER_SUB = 640
NEG_SLOPE = 0.01
NB = 1000
GRID_N = N // NB


def _sc_mesh():
    return plsc.VectorSubcoreMesh(core_axis_name="c", subcore_axis_name="s")


def _deg_body(e0s, e0d, e1s, e1d, e2s, e2d, e3s, e3d, out,
              cnt0, cnt1, cnt2, cnt3, cnt4, cnt5, cnt6, cnt7,
              idx_a, idx_b, sa, sb):
    c = lax.axis_index("c")
    s = lax.axis_index("s")
    wid = c * 16 + s
    cnts = (cnt0, cnt1, cnt2, cnt3, cnt4, cnt5, cnt6, cnt7)
    zeros16 = jnp.zeros((16,), jnp.float32)

    def zbody(i, carry):
        for cnt in cnts:
            cnt[pl.ds(i * 16, 16)] = zeros16
        return carry
    lax.fori_loop(0, N_PAD // 16, zbody, 0)

    ones16 = jnp.full((16,), 1.0, jnp.float32)
    streams = (e0s, e0d, e1s, e1d, e2s, e2d, e3s, e3d)
    per_sub = E // 32
    bufs = (idx_a, idx_b)
    sems = (sa, sb)

    def sl():
        return pl.ds(wid * per_sub, per_sub)

    pltpu.async_copy(streams[0].at[sl()], bufs[0], sems[0])
    for k in range(8):
        cur, csem = bufs[k % 2], sems[k % 2]
        pltpu.make_async_copy(streams[k].at[sl()], cur, csem).wait()
        if k < 7:
            pltpu.async_copy(streams[k + 1].at[sl()],
                             bufs[(k + 1) % 2], sems[(k + 1) % 2])

        def body(i, carry, k=k, cur=cur):
            idx16 = cur[pl.ds(i * 16, 16)]
            plsc.addupdate_scatter(cnts[k], [idx16], ones16)
            return carry
        lax.fori_loop(0, per_sub // 16, body, 0)

    for k in range(8):
        pltpu.sync_copy(cnts[k], out.at[pl.ds((wid * 8 + k) * N_PAD, N_PAD)])


def _degrees(srcs, dsts):
    fn = pl.kernel(
        _deg_body,
        out_type=jax.ShapeDtypeStruct((32 * 8 * N_PAD,), jnp.float32),
        mesh=_sc_mesh(),
        compiler_params=pltpu.CompilerParams(needs_layout_passes=False),
        scratch_types=[
            *([pltpu.VMEM((N_PAD,), jnp.float32)] * 8),
            pltpu.VMEM((E // 32,), jnp.int32),
            pltpu.VMEM((E // 32,), jnp.int32),
            pltpu.SemaphoreType.DMA,
            pltpu.SemaphoreType.DMA,
        ],
    )
    return fn(srcs[0], dsts[0], srcs[1], dsts[1], srcs[2], dsts[2],
              srcs[3], dsts[3])


def _degred_tc(x_ref, o_ref):
    acc = x_ref[0:8, :]
    for t in range(1, 32):
        acc = acc + x_ref[t * 8:(t + 1) * 8, :]
    o_ref[...] = acc


def _degred(deg_flat):
    x = deg_flat.reshape(32 * 8, N_PAD)
    return pl.pallas_call(
        _degred_tc,
        grid=(N_PAD // 128,),
        in_specs=[pl.BlockSpec((32 * 8, 128), lambda i: (0, i))],
        out_specs=pl.BlockSpec((8, 128), lambda i: (0, i)),
        out_shape=jax.ShapeDtypeStruct((8, N_PAD), jnp.float32),
    )(x)


AGG_CHUNK = 128
AGG_NCH = -(-(E // 32) // AGG_CHUNK)
AGG_PS = (AGG_NCH + 1) * AGG_CHUNK


def _agg_body(h, src, dst, zeros, out, acc,
              ix0, ix1, dx0, dx1, rows0, rows1, is0, is1, gs0, gs1):
    c = lax.axis_index("c")
    s = lax.axis_index("s")
    base = (c * 16 + s) * AGG_PS
    pltpu.sync_copy(zeros, acc.at[pl.ds(s * ROWS_PER_SUB, ROWS_PER_SUB)])
    plsc.subcore_barrier()

    ix = (ix0, ix1)
    dx = (dx0, dx1)
    rows = (rows0, rows1)
    isem = (is0, is1)
    gsem = (gs0, gs1)

    def islice(j):
        return pl.ds(base + j * AGG_CHUNK, AGG_CHUNK)

    def idx_start(slot, j):
        pltpu.async_copy(src.at[islice(j)], ix[slot], isem[slot])
        pltpu.async_copy(dst.at[islice(j)], dx[slot], isem[slot])

    def idx_wait(slot, j):
        pltpu.make_async_copy(src.at[islice(j)], ix[slot], isem[slot]).wait()
        pltpu.make_async_copy(dst.at[islice(j)], dx[slot], isem[slot]).wait()

    def g_start(slot):
        pltpu.async_copy(h.at[ix[slot]], rows[slot], gsem[slot])

    def fin(slot):
        pltpu.make_async_copy(h.at[ix[slot]], rows[slot], gsem[slot]).wait()
        pltpu.sync_copy(rows[slot], acc.at[dx[slot]], add=True)

    idx_start(0, 0)
    idx_wait(0, 0)
    g_start(0)
    idx_start(1, 1)

    def phase(j, slot):
        nxt = 1 - slot
        idx_wait(nxt, j + 1)
        g_start(nxt)
        fin(slot)
        idx_start(slot, j + 2)

    def body(i, carry):
        phase(2 * i, 0)
        phase(2 * i + 1, 1)
        return carry
    lax.fori_loop(0, (AGG_NCH - 1) // 2, body, 0)
    if AGG_NCH % 2 == 0:
        phase(AGG_NCH - 2, 0)
        fin(1)
        idx_wait(0, AGG_NCH)
    else:
        fin(0)
        idx_wait(1, AGG_NCH)

    plsc.subcore_barrier()
    pltpu.sync_copy(acc.at[pl.ds(s * ROWS_PER_SUB, ROWS_PER_SUB)],
                    out.at[c, pl.ds(s * ROWS_PER_SUB, ROWS_PER_SUB)])


def _aggregate(h, src_p, dst_p):
    zeros = jnp.zeros((ROWS_PER_SUB, D), jnp.float32)
    fn = pl.kernel(
        _agg_body,
        out_type=jax.ShapeDtypeStruct((2, N_PAD, D), jnp.float32),
        mesh=_sc_mesh(),
        scratch_types=[
            pltpu.VMEM_SHARED((N_PAD, D), jnp.float32),
            pltpu.VMEM((AGG_CHUNK,), jnp.int32),
            pltpu.VMEM((AGG_CHUNK,), jnp.int32),
            pltpu.VMEM((AGG_CHUNK,), jnp.int32),
            pltpu.VMEM((AGG_CHUNK,), jnp.int32),
            pltpu.VMEM((AGG_CHUNK, D), jnp.float32),
            pltpu.VMEM((AGG_CHUNK, D), jnp.float32),
            pltpu.SemaphoreType.DMA,
            pltpu.SemaphoreType.DMA,
            pltpu.SemaphoreType.DMA,
            pltpu.SemaphoreType.DMA,
        ],
    )
    return fn(h, src_p, dst_p, zeros)


def _prep_tc(deg_ref, f_ref, rs_ref, norm_ref, h0_0, h0_1, h0_2, h0_3):
    deg = deg_ref[...]
    norm = jnp.where(deg > 0.0, lax.rsqrt(jnp.maximum(deg, 1.0)), 0.0)
    norm_ref[...] = norm
    f = f_ref[...]
    for r, href in enumerate((h0_0, h0_1, h0_2, h0_3)):
        href[...] = f * rs_ref[r, :][None, :] * norm[:, 2 * r][:, None]


def _prep(degs, features, rs_pad):
    out_shape = (
        jax.ShapeDtypeStruct((N, 8), jnp.float32),
        *([jax.ShapeDtypeStruct((N, D), jnp.float32)] * 4),
    )
    return pl.pallas_call(
        _prep_tc,
        grid=(GRID_N,),
        in_specs=[
            pl.BlockSpec((NB, 8), lambda i: (i, 0)),
            pl.BlockSpec((NB, D), lambda i: (i, 0)),
            pl.BlockSpec((8, D), lambda i: (0, 0)),
        ],
        out_specs=(
            pl.BlockSpec((NB, 8), lambda i: (i, 0)),
            *([pl.BlockSpec((NB, D), lambda i: (i, 0))] * 4),
        ),
        out_shape=out_shape,
    )(degs, features, rs_pad)


def _dense_tc_stats(p_ref, nd_ref, w_ref, b_ref, y_ref, st_ref):
    agg = (p_ref[0] + p_ref[1]) * nd_ref[...]
    y = jnp.dot(agg, w_ref[...], preferred_element_type=jnp.float32) \
        + b_ref[0, :][None, :]
    y_ref[...] = y

    @pl.when(pl.program_id(0) == 0)
    def _():
        st_ref[...] = jnp.zeros_like(st_ref)
    st_ref[0:1, :] += jnp.sum(y, axis=0, keepdims=True)
    st_ref[1:2, :] += jnp.sum(y * y, axis=0, keepdims=True)


def _dense_tc(p_ref, nd_ref, w_ref, b_ref, y_ref):
    agg = (p_ref[0] + p_ref[1]) * nd_ref[...]
    y_ref[...] = jnp.dot(agg, w_ref[...], preferred_element_type=jnp.float32) \
        + b_ref[0, :][None, :]


def _dense(parts, norm_dst, w, b, with_stats):
    in_specs = [
        pl.BlockSpec((2, NB, D), lambda i: (0, i, 0)),
        pl.BlockSpec((NB, 1), lambda i: (i, 0)),
        pl.BlockSpec((D, D), lambda i: (0, 0)),
        pl.BlockSpec((1, D), lambda i: (0, 0)),
    ]
    if with_stats:
        return pl.pallas_call(
            _dense_tc_stats,
            grid=(GRID_N,),
            in_specs=in_specs,
            out_specs=(
                pl.BlockSpec((NB, D), lambda i: (i, 0)),
                pl.BlockSpec((8, D), lambda i: (0, 0)),
            ),
            out_shape=(
                jax.ShapeDtypeStruct((N, D), jnp.float32),
                jax.ShapeDtypeStruct((8, D), jnp.float32),
            ),
        )(parts, norm_dst, w, b[None, :])
    return pl.pallas_call(
        _dense_tc,
        grid=(GRID_N,),
        in_specs=in_specs,
        out_specs=pl.BlockSpec((NB, D), lambda i: (i, 0)),
        out_shape=jax.ShapeDtypeStruct((N, D), jnp.float32),
    )(parts, norm_dst, w, b[None, :])


def _post_tc(y_ref, st_ref, f_ref, r1_ref, ns_ref, g_ref, bt_ref, h1_ref):
    mean = st_ref[0:1, :] / N
    var = st_ref[1:2, :] / N - mean * mean
    z = (y_ref[...] - mean) * lax.rsqrt(var + 1e-5) * g_ref[0:1, :] \
        + bt_ref[0:1, :]
    z = jnp.where(z >= 0.0, z, NEG_SLOPE * z)
    emb = f_ref[...] + z
    h1_ref[...] = emb * r1_ref[0:1, :] * ns_ref[...]


def _post(y, stats, features, r1_row, norm_src, gamma, beta):
    return pl.pallas_call(
        _post_tc,
        grid=(GRID_N,),
        in_specs=[
            pl.BlockSpec((NB, D), lambda i: (i, 0)),
            pl.BlockSpec((8, D), lambda i: (0, 0)),
            pl.BlockSpec((NB, D), lambda i: (i, 0)),
            pl.BlockSpec((1, D), lambda i: (0, 0)),
            pl.BlockSpec((NB, 1), lambda i: (i, 0)),
            pl.BlockSpec((1, D), lambda i: (0, 0)),
            pl.BlockSpec((1, D), lambda i: (0, 0)),
        ],
        out_specs=pl.BlockSpec((NB, D), lambda i: (i, 0)),
        out_shape=jax.ShapeDtypeStruct((N, D), jnp.float32),
    )(y, stats, features, r1_row, norm_src, gamma[None, :], beta[None, :])


def _relvec_tc(rs_ref, w0_ref, b0_ref, w1_ref, b1_ref, r1_ref, r2_ref):
    r1 = jnp.dot(rs_ref[...], w0_ref[...], preferred_element_type=jnp.float32) \
        + b0_ref[0:1, :]
    r2 = jnp.dot(r1, w1_ref[...], preferred_element_type=jnp.float32) \
        + b1_ref[0:1, :]
    r1_ref[...] = r1
    r2_ref[...] = r2


def _relvec(rs_pad, w0, b0, w1, b1):
    return pl.pallas_call(
        _relvec_tc,
        out_shape=(
            jax.ShapeDtypeStruct((8, D), jnp.float32),
            jax.ShapeDtypeStruct((8, D), jnp.float32),
        ),
    )(rs_pad, w0, b0[None, :], w1, b1[None, :])


def kernel(features, poi_r, s_r, d_r, n_r,
           poi_edge_index, s_edge_index, d_edge_index, n_edge_index,
           W_gcn0, b_gcn0, W_gcn1, b_gcn1,
           bn_gamma0, bn_beta0,
           W_rel0, b_rel0, W_rel1, b_rel1):
    edges = [n_edge_index, poi_edge_index, s_edge_index, d_edge_index]
    srcs = [e[0] for e in edges]
    dsts = [e[1] for e in edges]
    n_pad_e = AGG_PS - E // 32
    pad_pos = (jnp.arange(32)[:, None] * n_pad_e + jnp.arange(n_pad_e)[None, :])
    pad_src = (pad_pos % N).astype(jnp.int32)
    pad_dst = (N + pad_pos % (N_PAD - N)).astype(jnp.int32)

    def _pad_idx(x, pad_block):
        return jnp.concatenate(
            [x.reshape(32, E // 32), pad_block], axis=1).reshape(-1)
    srcs_p = [_pad_idx(x, pad_src) for x in srcs]
    dsts_p = [_pad_idx(x, pad_dst) for x in dsts]
    rs_pad = jnp.concatenate(
        [jnp.stack([n_r, poi_r, s_r, d_r]), jnp.zeros((4, D), jnp.float32)], axis=0)

    deg_flat = _degrees(srcs, dsts)
    deg8 = _degred(deg_flat)
    degs = deg8[:, :N].transpose(1, 0)

    r1_pad, r2_pad = _relvec(rs_pad, W_rel0, b_rel0, W_rel1, b_rel1)
    norms, h0_0, h0_1, h0_2, h0_3 = _prep(degs, features, rs_pad)
    h0s = (h0_0, h0_1, h0_2, h0_3)

    embs = []
    for r in range(4):
        nd = norms[:, 2 * r + 1:2 * r + 2]
        ns = norms[:, 2 * r:2 * r + 1]
        parts0 = _aggregate(h0s[r], srcs_p[r], dsts_p[r])
        y, stats = _dense(parts0, nd, W_gcn0, b_gcn0, True)
        h1 = _post(y, stats, features, r1_pad[r:r + 1, :], ns,
                   bn_gamma0, bn_beta0)
        parts1 = _aggregate(h1, srcs_p[r], dsts_p[r])
        embs.append(_dense(parts1, nd, W_gcn1, b_gcn1, False))

    return (embs[0], embs[1], embs[2], embs[3],
            r2_pad[0], r2_pad[1], r2_pad[2], r2_pad[3])
